# Initial kernel scaffold; baseline (speedup 1.0000x reference)
import functools

import jax
import jax.numpy as jnp
from jax import lax
from jax.experimental import pallas as pl
from jax.experimental.pallas import tpu as pltpu
from jax.experimental.pallas import tpu_sc as plsc

NC, NS = 2, 16
WC = 144
K = 80


def _leaky(v):
    return jnp.where(v >= 0, v, 0.15 * v)


def _edge_mlp_body(ea_ref, w_ref, b_ref, out_ref):
    v = jnp.dot(ea_ref[...], w_ref[...], preferred_element_type=jnp.float32)
    out_ref[...] = _leaky(v + b_ref[...])


def _edge_mlp(ea, Wcat, bcat, block_e=4000):
    E, ED = ea.shape
    W = Wcat.shape[1]
    return pl.pallas_call(
        _edge_mlp_body,
        grid=(E // block_e,),
        in_specs=[
            pl.BlockSpec((block_e, ED), lambda i: (i, 0)),
            pl.BlockSpec((ED, W), lambda i: (0, 0)),
            pl.BlockSpec((1, W), lambda i: (0, 0)),
        ],
        out_specs=pl.BlockSpec((block_e, W), lambda i: (i, 0)),
        out_shape=jax.ShapeDtypeStruct((E, W), jnp.float32),
    )(ea, Wcat, bcat)


def _sc_segsum(h1ext, dst, zeros_init):
    E = dst.shape[0]
    N = zeros_init.shape[0]
    ept = E // NS
    npt = N // NS
    nchunks = ept // K
    mesh = plsc.VectorSubcoreMesh(
        core_axis_name="c", subcore_axis_name="s",
        num_cores=NC, num_subcores=NS)

    @functools.partial(
        pl.kernel,
        out_type=jax.ShapeDtypeStruct((NC * N, WC), jnp.float32),
        mesh=mesh,
        scratch_types=[
            pltpu.VMEM((K,), jnp.int32),
            pltpu.VMEM((K, WC), jnp.float32),
            pltpu.VMEM_SHARED((N, WC), jnp.float32),
        ],
    )
    def body(h1_hbm, dst_hbm, zero_hbm, out_hbm, idx_v, rows_v, acc):
        c = lax.axis_index("c")
        s = lax.axis_index("s")
        r0 = s * npt
        pltpu.sync_copy(zero_hbm.at[pl.ds(r0, npt), :], acc.at[pl.ds(r0, npt), :])
        plsc.subcore_barrier()
        t0 = s * ept
        col0 = c * WC

        def chunk(i, carry):
            e0 = t0 + i * K
            pltpu.sync_copy(dst_hbm.at[pl.ds(e0, K)], idx_v)
            pltpu.sync_copy(h1_hbm.at[pl.ds(e0, K), pl.ds(col0, WC)], rows_v)
            pltpu.sync_copy(rows_v, acc.at[idx_v], add=True)
            return carry

        lax.fori_loop(0, nchunks, chunk, 0)
        plsc.subcore_barrier()
        pltpu.sync_copy(acc.at[pl.ds(r0, npt), :],
                        out_hbm.at[pl.ds(c * N + r0, npt), :])

    return body(h1ext, dst, zeros_init)


def _node_body(x_ref, s0_ref, s1_ref, batch_ref,
               c1nmW_ref, c1nmb_ref, c1e2W_ref, c1e2b_ref,
               c2nmW_ref, c2nmb_ref, c2e2W_ref, c2e2b_ref,
               muW_ref, mub_ref, lvW_ref, lvb_ref, clsW_ref, clsb_ref,
               z_ref, mu_ref, lv_ref, logit_ref, pooled_acc, cnt_acc):
    i = pl.program_id(0)
    nb = pl.num_programs(0)
    x = x_ref[...]
    H = c1nmb_ref.shape[1]
    D = x.shape[1]
    s0 = s0_ref[...]
    S1 = s0[:, :H]
    deg = s0[:, H:H + 1]
    S2 = s1_ref[...][:, :H]

    f32 = jnp.float32
    w1 = c1nmW_ref[...]
    A1, Be1 = w1[:D], w1[D:]
    W2_1 = jnp.dot(c1e2W_ref[...], Be1, preferred_element_type=f32)
    u1 = jnp.dot(c1e2b_ref[...], Be1, preferred_element_type=f32) + c1nmb_ref[...]
    h = _leaky(deg * (jnp.dot(x, A1, preferred_element_type=f32) + u1)
               + jnp.dot(S1, W2_1, preferred_element_type=f32))

    w2 = c2nmW_ref[...]
    A2, Be2 = w2[:H], w2[H:]
    W2_2 = jnp.dot(c2e2W_ref[...], Be2, preferred_element_type=f32)
    u2 = jnp.dot(c2e2b_ref[...], Be2, preferred_element_type=f32) + c2nmb_ref[...]
    h2 = _leaky(deg * (jnp.dot(h, A2, preferred_element_type=f32) + u2)
                + jnp.dot(S2, W2_2, preferred_element_type=f32))

    mu = jnp.dot(h2, muW_ref[...], preferred_element_type=f32) + mub_ref[...]
    lv = jnp.dot(h2, lvW_ref[...], preferred_element_type=f32) + lvb_ref[...]
    z_ref[...] = mu
    mu_ref[...] = mu
    lv_ref[...] = lv

    G = pooled_acc.shape[0]
    rb = x.shape[0]
    b = batch_ref[0, 0, :]
    gids = lax.broadcasted_iota(jnp.int32, (rb, G), 1)
    oh = (b[:, None] == gids).astype(f32)

    @pl.when(i == 0)
    def _():
        pooled_acc[...] = jnp.zeros_like(pooled_acc)
        cnt_acc[...] = jnp.zeros_like(cnt_acc)

    pooled_acc[...] += lax.dot_general(
        oh, mu, (((0,), (0,)), ((), ())), preferred_element_type=f32)
    cnt_acc[...] += lax.dot_general(
        oh, jnp.ones((rb, 128), f32), (((0,), (0,)), ((), ())),
        preferred_element_type=f32)

    @pl.when(i == nb - 1)
    def _():
        cnt = jnp.maximum(cnt_acc[:, 0:1], 1.0)
        pooled = pooled_acc[...] / cnt
        logit_ref[...] = (jnp.dot(pooled, clsW_ref[...],
                                  preferred_element_type=f32)
                          + clsb_ref[...])


def _node_stage(x, S0, S1, batch_r,
                c1nmW, c1nmb, c1e2W, c1e2b, c2nmW, c2nmb, c2e2W, c2e2b,
                muW, mub, lvW, lvb, clsW, clsb, block_n=1000):
    N, D = x.shape
    L = muW.shape[1]
    G, C = clsW.shape[0], clsW.shape[1]
    nb = N // block_n
    row = lambda i: (i, 0)
    cst = lambda i: (0, 0)
    full = lambda a: pl.BlockSpec(a.shape, cst)
    out = pl.pallas_call(
        _node_body,
        grid=(nb,),
        in_specs=[
            pl.BlockSpec((block_n, D), row),
            pl.BlockSpec((block_n, WC), row),
            pl.BlockSpec((block_n, WC), row),
            pl.BlockSpec((1, 1, block_n), lambda i: (i, 0, 0)),
            full(c1nmW), full(c1nmb), full(c1e2W), full(c1e2b),
            full(c2nmW), full(c2nmb), full(c2e2W), full(c2e2b),
            full(muW), full(mub), full(lvW), full(lvb),
            full(clsW), full(clsb),
        ],
        out_specs=[
            pl.BlockSpec((block_n, L), row),
            pl.BlockSpec((block_n, L), row),
            pl.BlockSpec((block_n, L), row),
            pl.BlockSpec((G, C), cst),
        ],
        out_shape=[
            jax.ShapeDtypeStruct((N, L), jnp.float32),
            jax.ShapeDtypeStruct((N, L), jnp.float32),
            jax.ShapeDtypeStruct((N, L), jnp.float32),
            jax.ShapeDtypeStruct((G, C), jnp.float32),
        ],
        scratch_shapes=[
            pltpu.VMEM((G, L), jnp.float32),
            pltpu.VMEM((G, 128), jnp.float32),
        ],
    )(x, S0, S1, batch_r,
      c1nmW, c1nmb, c1e2W, c1e2b, c2nmW, c2nmb, c2e2W, c2e2b,
      muW, mub, lvW, lvb, clsW, clsb)
    return out


def kernel(x, edge_index, edge_attr, batch, eps,
           c1e1W, c1e1b, c1e2W, c1e2b, c1nmW, c1nmb,
           c2e1W, c2e1b, c2e2W, c2e2b, c2nmW, c2nmb,
           muW, mub, lvW, lvb, clsW, clsb):
    N, D = x.shape
    E, ED = edge_attr.shape
    H = c1e1W.shape[1]
    dst = edge_index[1]

    Wcat = jnp.zeros((ED, 2 * WC), jnp.float32)
    Wcat = Wcat.at[:, :H].set(c1e1W).at[:, WC:WC + H].set(c2e1W)
    bcat = jnp.zeros((1, 2 * WC), jnp.float32)
    bcat = bcat.at[0, :H].set(c1e1b).at[0, H].set(1.0)
    bcat = bcat.at[0, WC:WC + H].set(c2e1b)

    h1ext = _edge_mlp(edge_attr, Wcat, bcat)
    S = _sc_segsum(h1ext, dst, jnp.zeros((N, WC), jnp.float32))
    S0, S1 = S[:N], S[N:]

    batch_r = batch.reshape(10, 1, N // 10)
    z, mu, lv, logits = _node_stage(
        x, S0, S1, batch_r,
        c1nmW, c1nmb.reshape(1, H), c1e2W, c1e2b.reshape(1, H),
        c2nmW, c2nmb.reshape(1, H), c2e2W, c2e2b.reshape(1, H),
        muW, mub.reshape(1, -1), lvW, lvb.reshape(1, -1),
        clsW, clsb.reshape(1, -1))
    return (z, mu, lv, logits)


# speedup vs baseline: 3.9467x; 3.9467x over previous
import functools

import jax
import jax.numpy as jnp
from jax import lax
from jax.experimental import pallas as pl
from jax.experimental.pallas import tpu as pltpu
from jax.experimental.pallas import tpu_sc as plsc

NC, NS = 2, 16
WC = 128
K = 80


def _leaky(v):
    return jnp.where(v >= 0, v, 0.15 * v)


def _edge_mlp_body(ea_ref, w_ref, b_ref, dst_ref, out_ref, deg_ref, deg_acc):
    i = pl.program_id(0)
    nb = pl.num_programs(0)
    f32 = jnp.float32
    v = jnp.dot(ea_ref[...], w_ref[...], preferred_element_type=f32)
    v = _leaky(v + b_ref[...])
    out_ref[0, :, :] = v[:, :WC]
    out_ref[1, :, :] = v[:, WC:]

    d = dst_ref[0, 0, :]
    be = d.shape[0]
    nh = deg_acc.shape[0]
    hi = d >> 7
    lo = d & 127
    oh_hi = (hi[:, None] == lax.broadcasted_iota(jnp.int32, (be, nh), 1)).astype(f32)
    oh_lo = (lo[:, None] == lax.broadcasted_iota(jnp.int32, (be, 128), 1)).astype(f32)

    @pl.when(i == 0)
    def _():
        deg_acc[...] = jnp.zeros_like(deg_acc)

    deg_acc[...] += lax.dot_general(
        oh_hi, oh_lo, (((0,), (0,)), ((), ())), preferred_element_type=f32)

    @pl.when(i == nb - 1)
    def _():
        deg_ref[...] = deg_acc[...]


def _edge_mlp(ea, Wcat, bcat, dst, nh, block_e=2000):
    E, ED = ea.shape
    W = Wcat.shape[1]
    nb = E // block_e
    return pl.pallas_call(
        _edge_mlp_body,
        grid=(nb,),
        in_specs=[
            pl.BlockSpec((block_e, ED), lambda i: (i, 0)),
            pl.BlockSpec((ED, W), lambda i: (0, 0)),
            pl.BlockSpec((1, W), lambda i: (0, 0)),
            pl.BlockSpec((1, 1, block_e), lambda i: (i, 0, 0)),
        ],
        out_specs=[
            pl.BlockSpec((NC, block_e, WC), lambda i: (0, i, 0)),
            pl.BlockSpec((nh, 128), lambda i: (0, 0)),
        ],
        out_shape=[
            jax.ShapeDtypeStruct((NC, E, WC), jnp.float32),
            jax.ShapeDtypeStruct((nh, 128), jnp.float32),
        ],
        scratch_shapes=[pltpu.VMEM((nh, 128), jnp.float32)],
    )(ea, Wcat, bcat, dst.reshape(nb, 1, block_e))


def _sc_segsum(h1ext, dst, zeros_init):
    E = dst.shape[0]
    Np = zeros_init.shape[0]
    ept = E // NS
    npt = Np // NS
    nchunks = ept // K
    mesh = plsc.VectorSubcoreMesh(
        core_axis_name="c", subcore_axis_name="s",
        num_cores=NC, num_subcores=NS)

    @functools.partial(
        pl.kernel,
        out_type=jax.ShapeDtypeStruct((NC * Np, WC), jnp.float32),
        mesh=mesh,
        scratch_types=[
            pltpu.VMEM((K,), jnp.int32),
            pltpu.VMEM((K, WC), jnp.float32),
            pltpu.VMEM_SHARED((Np, WC), jnp.float32),
        ],
    )
    def body(h1_hbm, dst_hbm, zero_hbm, out_hbm, idx_v, rows_v, acc):
        c = lax.axis_index("c")
        s = lax.axis_index("s")
        r0 = s * npt
        pltpu.sync_copy(zero_hbm.at[pl.ds(r0, npt), :], acc.at[pl.ds(r0, npt), :])
        plsc.subcore_barrier()
        t0 = s * ept

        def chunk(i, carry):
            e0 = t0 + i * K
            pltpu.sync_copy(dst_hbm.at[pl.ds(e0, K)], idx_v)
            pltpu.sync_copy(h1_hbm.at[c, pl.ds(e0, K), :], rows_v)
            pltpu.sync_copy(rows_v, acc.at[idx_v], add=True)
            return carry

        lax.fori_loop(0, nchunks, chunk, 0)
        plsc.subcore_barrier()
        pltpu.sync_copy(acc.at[pl.ds(r0, npt), :],
                        out_hbm.at[pl.ds(c * Np + r0, npt), :])

    return body(h1ext, dst, zeros_init)


def _node_body(x_ref, s0_ref, s1_ref, deg_ref, batch_ref,
               c1nmW_ref, c1nmb_ref, c1e2W_ref, c1e2b_ref,
               c2nmW_ref, c2nmb_ref, c2e2W_ref, c2e2b_ref,
               muW_ref, mub_ref, lvW_ref, lvb_ref, clsW_ref, clsb_ref,
               z_ref, mu_ref, lv_ref, logit_ref, pooled_acc, cnt_acc):
    i = pl.program_id(0)
    nb = pl.num_programs(0)
    x = x_ref[...]
    H = c1nmb_ref.shape[1]
    D = x.shape[1]
    S1 = s0_ref[...]
    S2 = s1_ref[...]
    deg = deg_ref[...]

    f32 = jnp.float32
    w1 = c1nmW_ref[...]
    A1, Be1 = w1[:D], w1[D:]
    W2_1 = jnp.dot(c1e2W_ref[...], Be1, preferred_element_type=f32)
    u1 = jnp.dot(c1e2b_ref[...], Be1, preferred_element_type=f32) + c1nmb_ref[...]
    h = _leaky(deg * (jnp.dot(x, A1, preferred_element_type=f32) + u1)
               + jnp.dot(S1, W2_1, preferred_element_type=f32))

    w2 = c2nmW_ref[...]
    A2, Be2 = w2[:H], w2[H:]
    W2_2 = jnp.dot(c2e2W_ref[...], Be2, preferred_element_type=f32)
    u2 = jnp.dot(c2e2b_ref[...], Be2, preferred_element_type=f32) + c2nmb_ref[...]
    h2 = _leaky(deg * (jnp.dot(h, A2, preferred_element_type=f32) + u2)
                + jnp.dot(S2, W2_2, preferred_element_type=f32))

    mu = jnp.dot(h2, muW_ref[...], preferred_element_type=f32) + mub_ref[...]
    lv = jnp.dot(h2, lvW_ref[...], preferred_element_type=f32) + lvb_ref[...]
    z_ref[...] = mu
    mu_ref[...] = mu
    lv_ref[...] = lv

    G = pooled_acc.shape[0]
    rb = x.shape[0]
    b = batch_ref[0, 0, :]
    gids = lax.broadcasted_iota(jnp.int32, (rb, G), 1)
    oh = (b[:, None] == gids).astype(f32)

    @pl.when(i == 0)
    def _():
        pooled_acc[...] = jnp.zeros_like(pooled_acc)
        cnt_acc[...] = jnp.zeros_like(cnt_acc)

    pooled_acc[...] += lax.dot_general(
        oh, mu, (((0,), (0,)), ((), ())), preferred_element_type=f32)
    cnt_acc[...] += lax.dot_general(
        oh, jnp.ones((rb, 128), f32), (((0,), (0,)), ((), ())),
        preferred_element_type=f32)

    @pl.when(i == nb - 1)
    def _():
        cnt = jnp.maximum(cnt_acc[:, 0:1], 1.0)
        pooled = pooled_acc[...] / cnt
        logit_ref[...] = (jnp.dot(pooled, clsW_ref[...],
                                  preferred_element_type=f32)
                          + clsb_ref[...])


def _node_stage(x, S0, S1, deg, batch,
                c1nmW, c1nmb, c1e2W, c1e2b, c2nmW, c2nmb, c2e2W, c2e2b,
                muW, mub, lvW, lvb, clsW, clsb, block_n=1000):
    N, D = x.shape
    L = muW.shape[1]
    G, C = clsW.shape[0], clsW.shape[1]
    nb = N // block_n
    batch_r = batch.reshape(nb, 1, block_n)
    row = lambda i: (i, 0)
    cst = lambda i: (0, 0)
    full = lambda a: pl.BlockSpec(a.shape, cst)
    out = pl.pallas_call(
        _node_body,
        grid=(nb,),
        in_specs=[
            pl.BlockSpec((block_n, D), row),
            pl.BlockSpec((block_n, WC), row),
            pl.BlockSpec((block_n, WC), row),
            pl.BlockSpec((block_n, 1), row),
            pl.BlockSpec((1, 1, block_n), lambda i: (i, 0, 0)),
            full(c1nmW), full(c1nmb), full(c1e2W), full(c1e2b),
            full(c2nmW), full(c2nmb), full(c2e2W), full(c2e2b),
            full(muW), full(mub), full(lvW), full(lvb),
            full(clsW), full(clsb),
        ],
        out_specs=[
            pl.BlockSpec((block_n, L), row),
            pl.BlockSpec((block_n, L), row),
            pl.BlockSpec((block_n, L), row),
            pl.BlockSpec((G, C), cst),
        ],
        out_shape=[
            jax.ShapeDtypeStruct((N, L), jnp.float32),
            jax.ShapeDtypeStruct((N, L), jnp.float32),
            jax.ShapeDtypeStruct((N, L), jnp.float32),
            jax.ShapeDtypeStruct((G, C), jnp.float32),
        ],
        scratch_shapes=[
            pltpu.VMEM((G, L), jnp.float32),
            pltpu.VMEM((G, 128), jnp.float32),
        ],
    )(x, S0, S1, deg, batch_r,
      c1nmW, c1nmb, c1e2W, c1e2b, c2nmW, c2nmb, c2e2W, c2e2b,
      muW, mub, lvW, lvb, clsW, clsb)
    return out


def kernel(x, edge_index, edge_attr, batch, eps,
           c1e1W, c1e1b, c1e2W, c1e2b, c1nmW, c1nmb,
           c2e1W, c2e1b, c2e2W, c2e2b, c2nmW, c2nmb,
           muW, mub, lvW, lvb, clsW, clsb):
    N, D = x.shape
    E, ED = edge_attr.shape
    H = c1e1W.shape[1]
    dst = edge_index[1]

    Wcat = jnp.concatenate([c1e1W, c2e1W], axis=1).astype(jnp.float32)
    bcat = jnp.concatenate([c1e1b, c2e1b]).reshape(1, 2 * H).astype(jnp.float32)

    nh = (N + 127) // 128
    h1ext, deg80 = _edge_mlp(edge_attr, Wcat, bcat, dst, nh)
    deg = deg80.reshape(-1)[:N].reshape(N, 1)

    npad = 8 * NS
    Np = (N + npad - 1) // npad * npad
    S = _sc_segsum(h1ext, dst, jnp.zeros((Np, WC), jnp.float32))
    S0, S1 = S[:N], S[Np:Np + N]

    z, mu, lv, logits = _node_stage(
        x, S0, S1, deg, batch,
        c1nmW, c1nmb.reshape(1, H), c1e2W, c1e2b.reshape(1, H),
        c2nmW, c2nmb.reshape(1, H), c2e2W, c2e2b.reshape(1, H),
        muW, mub.reshape(1, -1), lvW, lvb.reshape(1, -1),
        clsW, clsb.reshape(1, -1))
    return (z, mu, lv, logits)


# speedup vs baseline: 7.1060x; 1.8005x over previous
import functools

import jax
import jax.numpy as jnp
from jax import lax
from jax.experimental import pallas as pl
from jax.experimental.pallas import tpu as pltpu
from jax.experimental.pallas import tpu_sc as plsc

NC, NS = 2, 16
WC = 128
K = 80


def _leaky(v):
    return jnp.where(v >= 0, v, 0.15 * v)


def _edge_mlp_body(ea_ref, w_ref, b_ref, dst_ref, out_ref, deg_ref, deg_acc):
    i = pl.program_id(0)
    nb = pl.num_programs(0)
    f32 = jnp.float32
    v = lax.dot_general(ea_ref[...], w_ref[...], (((0,), (0,)), ((), ())),
                        preferred_element_type=f32)
    v = _leaky(v + b_ref[...])
    out_ref[0, :, :] = v[:, :WC]
    out_ref[1, :, :] = v[:, WC:]

    d = dst_ref[0, 0, :]
    be = d.shape[0]
    nh = deg_acc.shape[0]
    hi = d >> 7
    lo = d & 127
    oh_hi = (hi[:, None] == lax.broadcasted_iota(jnp.int32, (be, nh), 1)).astype(f32)
    oh_lo = (lo[:, None] == lax.broadcasted_iota(jnp.int32, (be, 128), 1)).astype(f32)

    @pl.when(i == 0)
    def _():
        deg_acc[...] = jnp.zeros_like(deg_acc)

    deg_acc[...] += lax.dot_general(
        oh_hi, oh_lo, (((0,), (0,)), ((), ())), preferred_element_type=f32)

    @pl.when(i == nb - 1)
    def _():
        deg_ref[...] = deg_acc[...]


def _edge_mlp(ea_t, Wcat, bcat, dst, nh, block_e=2560):
    ED, E = ea_t.shape
    W = Wcat.shape[1]
    nb = E // block_e
    return pl.pallas_call(
        _edge_mlp_body,
        grid=(nb,),
        in_specs=[
            pl.BlockSpec((ED, block_e), lambda i: (0, i)),
            pl.BlockSpec((ED, W), lambda i: (0, 0)),
            pl.BlockSpec((1, W), lambda i: (0, 0)),
            pl.BlockSpec((1, 1, block_e), lambda i: (i, 0, 0)),
        ],
        out_specs=[
            pl.BlockSpec((NC, block_e, WC), lambda i: (0, i, 0)),
            pl.BlockSpec((nh, 128), lambda i: (0, 0)),
        ],
        out_shape=[
            jax.ShapeDtypeStruct((NC, E, WC), jnp.float32),
            jax.ShapeDtypeStruct((nh, 128), jnp.float32),
        ],
        scratch_shapes=[pltpu.VMEM((nh, 128), jnp.float32)],
    )(ea_t, Wcat, bcat, dst.reshape(nb, 1, block_e))


def _sc_segsum(h1ext, dst, zeros_init):
    E = dst.shape[0]
    Np = zeros_init.shape[0]
    ept = E // NS
    npt = Np // NS
    nchunks = ept // K
    mesh = plsc.VectorSubcoreMesh(
        core_axis_name="c", subcore_axis_name="s",
        num_cores=NC, num_subcores=NS)

    nbuf = 2
    assert nchunks % nbuf == 0

    @functools.partial(
        pl.kernel,
        out_type=jax.ShapeDtypeStruct((NC * Np, WC), jnp.float32),
        mesh=mesh,
        scratch_types=[
            pltpu.VMEM((nbuf, K), jnp.int32),
            pltpu.VMEM((nbuf, K, WC), jnp.float32),
            pltpu.VMEM_SHARED((Np, WC), jnp.float32),
            pltpu.SemaphoreType.DMA((nbuf,)),
            pltpu.SemaphoreType.DMA((nbuf,)),
        ],
    )
    def body(h1_hbm, dst_hbm, zero_hbm, out_hbm, idx_v, rows_v, acc,
             isem, rsem):
        c = lax.axis_index("c")
        s = lax.axis_index("s")
        r0 = s * npt
        pltpu.sync_copy(zero_hbm.at[pl.ds(r0, npt), :], acc.at[pl.ds(r0, npt), :])
        plsc.subcore_barrier()
        t0 = s * ept

        def load(e0, b):
            pltpu.async_copy(dst_hbm.at[pl.ds(e0, K)], idx_v.at[b],
                             isem.at[b])
            pltpu.async_copy(h1_hbm.at[c, pl.ds(e0, K), :], rows_v.at[b],
                             rsem.at[b])

        for b in range(nbuf):
            load(t0 + b * K, b)

        def outer(g, carry):
            for b in range(nbuf):
                i = g * nbuf + b
                e0 = t0 + i * K
                pltpu.make_async_copy(dst_hbm.at[pl.ds(0, K)], idx_v.at[b],
                                      isem.at[b]).wait()
                pltpu.make_async_copy(h1_hbm.at[0, pl.ds(0, K), :],
                                      rows_v.at[b], rsem.at[b]).wait()
                pltpu.sync_copy(rows_v.at[b], acc.at[idx_v.at[b]], add=True)

                @pl.when(i + nbuf < nchunks)
                def _():
                    load(e0 + nbuf * K, b)
            return carry

        lax.fori_loop(0, nchunks // nbuf, outer, 0)
        plsc.subcore_barrier()
        pltpu.sync_copy(acc.at[pl.ds(r0, npt), :],
                        out_hbm.at[pl.ds(c * Np + r0, npt), :])

    return body(h1ext, dst, zeros_init)


def _node_body(x_ref, s0_ref, s1_ref, deg_ref, batch_ref,
               c1nmW_ref, c1nmb_ref, c1e2W_ref, c1e2b_ref,
               c2nmW_ref, c2nmb_ref, c2e2W_ref, c2e2b_ref,
               muW_ref, mub_ref, lvW_ref, lvb_ref, clsW_ref, clsb_ref,
               z_ref, mu_ref, lv_ref, logit_ref, pooled_acc, cnt_acc):
    i = pl.program_id(0)
    nb = pl.num_programs(0)
    x = x_ref[...]
    H = c1nmb_ref.shape[1]
    D = x.shape[1]
    S1 = s0_ref[...]
    S2 = s1_ref[...]
    deg = deg_ref[...]

    f32 = jnp.float32
    w1 = c1nmW_ref[...]
    A1, Be1 = w1[:D], w1[D:]
    W2_1 = jnp.dot(c1e2W_ref[...], Be1, preferred_element_type=f32)
    u1 = jnp.dot(c1e2b_ref[...], Be1, preferred_element_type=f32) + c1nmb_ref[...]
    h = _leaky(deg * (jnp.dot(x, A1, preferred_element_type=f32) + u1)
               + jnp.dot(S1, W2_1, preferred_element_type=f32))

    w2 = c2nmW_ref[...]
    A2, Be2 = w2[:H], w2[H:]
    W2_2 = jnp.dot(c2e2W_ref[...], Be2, preferred_element_type=f32)
    u2 = jnp.dot(c2e2b_ref[...], Be2, preferred_element_type=f32) + c2nmb_ref[...]
    h2 = _leaky(deg * (jnp.dot(h, A2, preferred_element_type=f32) + u2)
                + jnp.dot(S2, W2_2, preferred_element_type=f32))

    mu = jnp.dot(h2, muW_ref[...], preferred_element_type=f32) + mub_ref[...]
    lv = jnp.dot(h2, lvW_ref[...], preferred_element_type=f32) + lvb_ref[...]
    z_ref[...] = mu
    mu_ref[...] = mu
    lv_ref[...] = lv

    G = pooled_acc.shape[0]
    rb = x.shape[0]
    b = batch_ref[0, 0, :]
    gids = lax.broadcasted_iota(jnp.int32, (rb, G), 1)
    oh = (b[:, None] == gids).astype(f32)

    @pl.when(i == 0)
    def _():
        pooled_acc[...] = jnp.zeros_like(pooled_acc)
        cnt_acc[...] = jnp.zeros_like(cnt_acc)

    pooled_acc[...] += lax.dot_general(
        oh, mu, (((0,), (0,)), ((), ())), preferred_element_type=f32)
    cnt_acc[...] += lax.dot_general(
        oh, jnp.ones((rb, 128), f32), (((0,), (0,)), ((), ())),
        preferred_element_type=f32)

    @pl.when(i == nb - 1)
    def _():
        cnt = jnp.maximum(cnt_acc[:, 0:1], 1.0)
        pooled = pooled_acc[...] / cnt
        logit_ref[...] = (jnp.dot(pooled, clsW_ref[...],
                                  preferred_element_type=f32)
                          + clsb_ref[...])


def _node_stage(x, S0, S1, deg, batch,
                c1nmW, c1nmb, c1e2W, c1e2b, c2nmW, c2nmb, c2e2W, c2e2b,
                muW, mub, lvW, lvb, clsW, clsb, block_n=1000):
    N, D = x.shape
    L = muW.shape[1]
    G, C = clsW.shape[0], clsW.shape[1]
    nb = N // block_n
    batch_r = batch.reshape(nb, 1, block_n)
    row = lambda i: (i, 0)
    cst = lambda i: (0, 0)
    full = lambda a: pl.BlockSpec(a.shape, cst)
    out = pl.pallas_call(
        _node_body,
        grid=(nb,),
        in_specs=[
            pl.BlockSpec((block_n, D), row),
            pl.BlockSpec((block_n, WC), row),
            pl.BlockSpec((block_n, WC), row),
            pl.BlockSpec((block_n, 1), row),
            pl.BlockSpec((1, 1, block_n), lambda i: (i, 0, 0)),
            full(c1nmW), full(c1nmb), full(c1e2W), full(c1e2b),
            full(c2nmW), full(c2nmb), full(c2e2W), full(c2e2b),
            full(muW), full(mub), full(lvW), full(lvb),
            full(clsW), full(clsb),
        ],
        out_specs=[
            pl.BlockSpec((block_n, L), row),
            pl.BlockSpec((block_n, L), row),
            pl.BlockSpec((block_n, L), row),
            pl.BlockSpec((G, C), cst),
        ],
        out_shape=[
            jax.ShapeDtypeStruct((N, L), jnp.float32),
            jax.ShapeDtypeStruct((N, L), jnp.float32),
            jax.ShapeDtypeStruct((N, L), jnp.float32),
            jax.ShapeDtypeStruct((G, C), jnp.float32),
        ],
        scratch_shapes=[
            pltpu.VMEM((G, L), jnp.float32),
            pltpu.VMEM((G, 128), jnp.float32),
        ],
    )(x, S0, S1, deg, batch_r,
      c1nmW, c1nmb, c1e2W, c1e2b, c2nmW, c2nmb, c2e2W, c2e2b,
      muW, mub, lvW, lvb, clsW, clsb)
    return out


def kernel(x, edge_index, edge_attr, batch, eps,
           c1e1W, c1e1b, c1e2W, c1e2b, c1nmW, c1nmb,
           c2e1W, c2e1b, c2e2W, c2e2b, c2nmW, c2nmb,
           muW, mub, lvW, lvb, clsW, clsb):
    N, D = x.shape
    E, ED = edge_attr.shape
    H = c1e1W.shape[1]
    dst = edge_index[1]

    Wcat = jnp.concatenate([c1e1W, c2e1W], axis=1).astype(jnp.float32)
    bcat = jnp.concatenate([c1e1b, c2e1b]).reshape(1, 2 * H).astype(jnp.float32)

    nh = (N + 127) // 128
    h1ext, deg80 = _edge_mlp(edge_attr.T, Wcat, bcat, dst, nh)
    deg = deg80.reshape(-1)[:N].reshape(N, 1)

    npad = 8 * NS
    Np = (N + npad - 1) // npad * npad
    S = _sc_segsum(h1ext, dst, jnp.zeros((Np, WC), jnp.float32))
    S0, S1 = S[:N], S[Np:Np + N]

    z, mu, lv, logits = _node_stage(
        x, S0, S1, deg, batch,
        c1nmW, c1nmb.reshape(1, H), c1e2W, c1e2b.reshape(1, H),
        c2nmW, c2nmb.reshape(1, H), c2e2W, c2e2b.reshape(1, H),
        muW, mub.reshape(1, -1), lvW, lvb.reshape(1, -1),
        clsW, clsb.reshape(1, -1))
    return (z, mu, lv, logits)


# speedup vs baseline: 8.2080x; 1.1551x over previous
import functools

import jax
import jax.numpy as jnp
from jax import lax
from jax.experimental import pallas as pl
from jax.experimental.pallas import tpu as pltpu
from jax.experimental.pallas import tpu_sc as plsc

NC, NS = 2, 16
WC = 128
K = 80
NSPLIT = 2


def _leaky(v):
    return jnp.where(v >= 0, v, 0.15 * v)


def _edge_mlp_body(ea_ref, w_ref, b_ref, dst_ref, out_ref, deg_ref, deg_acc):
    i = pl.program_id(0)
    nb = pl.num_programs(0)
    f32 = jnp.float32
    v = lax.dot_general(ea_ref[...], w_ref[...], (((0,), (0,)), ((), ())),
                        preferred_element_type=f32)
    v = _leaky(v + b_ref[...])
    out_ref[0, :, :] = v[:, :WC]
    out_ref[1, :, :] = v[:, WC:]

    d = dst_ref[0, 0, :]
    be = d.shape[0]
    nh = deg_acc.shape[0]
    hi = d >> 7
    lo = d & 127
    oh_hi = (hi[:, None] == lax.broadcasted_iota(jnp.int32, (be, nh), 1)).astype(f32)
    oh_lo = (lo[:, None] == lax.broadcasted_iota(jnp.int32, (be, 128), 1)).astype(f32)

    @pl.when(i == 0)
    def _():
        deg_acc[...] = jnp.zeros_like(deg_acc)

    deg_acc[...] += lax.dot_general(
        oh_hi, oh_lo, (((0,), (0,)), ((), ())), preferred_element_type=f32)

    @pl.when(i == nb - 1)
    def _():
        deg_ref[...] = deg_acc[...]


def _edge_mlp(ea_t, Wcat, bcat, dst_r, nh, p, nsplit, block_e):
    ED, E = ea_t.shape
    Eh = E // nsplit
    W = Wcat.shape[1]
    nb = Eh // block_e
    off = p * nb
    return pl.pallas_call(
        _edge_mlp_body,
        grid=(nb,),
        in_specs=[
            pl.BlockSpec((ED, block_e), lambda i: (0, i + off)),
            pl.BlockSpec((ED, W), lambda i: (0, 0)),
            pl.BlockSpec((1, W), lambda i: (0, 0)),
            pl.BlockSpec((1, 1, block_e), lambda i: (i + off, 0, 0)),
        ],
        out_specs=[
            pl.BlockSpec((NC, block_e, WC), lambda i: (0, i, 0)),
            pl.BlockSpec((nh, 128), lambda i: (0, 0)),
        ],
        out_shape=[
            jax.ShapeDtypeStruct((NC, Eh, WC), jnp.float32),
            jax.ShapeDtypeStruct((nh, 128), jnp.float32),
        ],
        scratch_shapes=[pltpu.VMEM((nh, 128), jnp.float32)],
    )(ea_t, Wcat, bcat, dst_r)


def _sc_segsum(h1p, dst, zeros_init, eoff):
    Eh = h1p.shape[1]
    Np = zeros_init.shape[0]
    ept = Eh // NS
    npt = Np // NS
    nchunks = ept // K
    nbuf = 2
    mesh = plsc.VectorSubcoreMesh(
        core_axis_name="c", subcore_axis_name="s",
        num_cores=NC, num_subcores=NS)

    @functools.partial(
        pl.kernel,
        out_type=[jax.ShapeDtypeStruct((Np, WC), jnp.float32),
                  jax.ShapeDtypeStruct((Np, WC), jnp.float32)],
        mesh=mesh,
        scratch_types=[
            pltpu.VMEM((nbuf, K), jnp.int32),
            pltpu.VMEM((nbuf, K, WC), jnp.float32),
            pltpu.VMEM_SHARED((Np, WC), jnp.float32),
            pltpu.SemaphoreType.DMA((nbuf,)),
            pltpu.SemaphoreType.DMA((nbuf,)),
        ],
    )
    def body(h1_hbm, dst_hbm, zero_hbm, out0_hbm, out1_hbm, idx_v, rows_v,
             acc, isem, rsem):
        c = lax.axis_index("c")
        s = lax.axis_index("s")
        r0 = s * npt
        pltpu.sync_copy(zero_hbm.at[pl.ds(r0, npt), :], acc.at[pl.ds(r0, npt), :])
        plsc.subcore_barrier()
        t0 = s * ept

        def load(e0, b):
            pltpu.async_copy(dst_hbm.at[pl.ds(eoff + e0, K)], idx_v.at[b],
                             isem.at[b])
            pltpu.async_copy(h1_hbm.at[c, pl.ds(e0, K), :], rows_v.at[b],
                             rsem.at[b])

        def consume(i, e0, b):
            pltpu.make_async_copy(dst_hbm.at[pl.ds(0, K)], idx_v.at[b],
                                  isem.at[b]).wait()
            pltpu.make_async_copy(h1_hbm.at[0, pl.ds(0, K), :],
                                  rows_v.at[b], rsem.at[b]).wait()
            pltpu.sync_copy(rows_v.at[b], acc.at[idx_v.at[b]], add=True)

            @pl.when(i + nbuf < nchunks)
            def _():
                load(e0 + nbuf * K, b)

        for b in range(min(nbuf, nchunks)):
            load(t0 + b * K, b)

        def outer(g, carry):
            for b in range(nbuf):
                i = g * nbuf + b
                consume(i, t0 + i * K, b)
            return carry

        lax.fori_loop(0, nchunks // nbuf, outer, 0)
        for b in range(nchunks % nbuf):
            i = (nchunks // nbuf) * nbuf + b
            consume(i, t0 + i * K, b)

        plsc.subcore_barrier()

        @pl.when(c == 0)
        def _():
            pltpu.sync_copy(acc.at[pl.ds(r0, npt), :],
                            out0_hbm.at[pl.ds(r0, npt), :])

        @pl.when(c == 1)
        def _():
            pltpu.sync_copy(acc.at[pl.ds(r0, npt), :],
                            out1_hbm.at[pl.ds(r0, npt), :])

    return body(h1p, dst, zeros_init)


def _node_body(x_ref, s0_refs, s1_refs, deg_refs, batch_ref,
               c1nmW_ref, c1nmb_ref, c1e2W_ref, c1e2b_ref,
               c2nmW_ref, c2nmb_ref, c2e2W_ref, c2e2b_ref,
               muW_ref, mub_ref, lvW_ref, lvb_ref, clsW_ref, clsb_ref,
               z_ref, mu_ref, lv_ref, logit_ref, pooled_acc, cnt_acc):
    i = pl.program_id(0)
    nb = pl.num_programs(0)
    x = x_ref[...]
    H = c1nmb_ref.shape[1]
    D = x.shape[1]
    S1 = s0_refs[0][...]
    for r in s0_refs[1:]:
        S1 = S1 + r[...]
    S2 = s1_refs[0][...]
    for r in s1_refs[1:]:
        S2 = S2 + r[...]
    deg = deg_refs[0][...]
    for r in deg_refs[1:]:
        deg = deg + r[...]

    f32 = jnp.float32
    w1 = c1nmW_ref[...]
    A1, Be1 = w1[:D], w1[D:]
    W2_1 = jnp.dot(c1e2W_ref[...], Be1, preferred_element_type=f32)
    u1 = jnp.dot(c1e2b_ref[...], Be1, preferred_element_type=f32) + c1nmb_ref[...]
    h = _leaky(deg * (jnp.dot(x, A1, preferred_element_type=f32) + u1)
               + jnp.dot(S1, W2_1, preferred_element_type=f32))

    w2 = c2nmW_ref[...]
    A2, Be2 = w2[:H], w2[H:]
    W2_2 = jnp.dot(c2e2W_ref[...], Be2, preferred_element_type=f32)
    u2 = jnp.dot(c2e2b_ref[...], Be2, preferred_element_type=f32) + c2nmb_ref[...]
    h2 = _leaky(deg * (jnp.dot(h, A2, preferred_element_type=f32) + u2)
                + jnp.dot(S2, W2_2, preferred_element_type=f32))

    mu = jnp.dot(h2, muW_ref[...], preferred_element_type=f32) + mub_ref[...]
    lv = jnp.dot(h2, lvW_ref[...], preferred_element_type=f32) + lvb_ref[...]
    z_ref[...] = mu
    mu_ref[...] = mu
    lv_ref[...] = lv

    G = pooled_acc.shape[0]
    rb = x.shape[0]
    b = batch_ref[0, 0, :]
    gids = lax.broadcasted_iota(jnp.int32, (rb, G), 1)
    oh = (b[:, None] == gids).astype(f32)

    @pl.when(i == 0)
    def _():
        pooled_acc[...] = jnp.zeros_like(pooled_acc)
        cnt_acc[...] = jnp.zeros_like(cnt_acc)

    pooled_acc[...] += lax.dot_general(
        oh, mu, (((0,), (0,)), ((), ())), preferred_element_type=f32)
    cnt_acc[...] += lax.dot_general(
        oh, jnp.ones((rb, 128), f32), (((0,), (0,)), ((), ())),
        preferred_element_type=f32)

    @pl.when(i == nb - 1)
    def _():
        cnt = jnp.maximum(cnt_acc[:, 0:1], 1.0)
        pooled = pooled_acc[...] / cnt
        logit_ref[...] = (jnp.dot(pooled, clsW_ref[...],
                                  preferred_element_type=f32)
                          + clsb_ref[...])


def _node_stage(x, S0s, S1s, degs, batch,
                c1nmW, c1nmb, c1e2W, c1e2b, c2nmW, c2nmb, c2e2W, c2e2b,
                muW, mub, lvW, lvb, clsW, clsb, block_n=2000):
    N, D = x.shape
    L = muW.shape[1]
    G, C = clsW.shape[0], clsW.shape[1]
    nb = N // block_n
    batch_r = batch.reshape(nb, 1, block_n)
    row = lambda i: (i, 0)
    cst = lambda i: (0, 0)
    full = lambda a: pl.BlockSpec(a.shape, cst)
    srow = pl.BlockSpec((block_n, WC), row)

    def wrapped(x_ref, *rest):
        np_ = len(S0s)
        s0r = rest[:np_]
        s1r = rest[np_:2 * np_]
        degr = rest[2 * np_:3 * np_]
        _node_body(x_ref, s0r, s1r, degr, *rest[3 * np_:])

    out = pl.pallas_call(
        wrapped,
        grid=(nb,),
        in_specs=[pl.BlockSpec((block_n, D), row)]
        + [srow] * len(S0s) + [srow] * len(S1s)
        + [pl.BlockSpec((block_n, 1), row)] * len(degs)
        + [pl.BlockSpec((1, 1, block_n), lambda i: (i, 0, 0)),
           full(c1nmW), full(c1nmb), full(c1e2W), full(c1e2b),
           full(c2nmW), full(c2nmb), full(c2e2W), full(c2e2b),
           full(muW), full(mub), full(lvW), full(lvb),
           full(clsW), full(clsb)],
        out_specs=[
            pl.BlockSpec((block_n, L), row),
            pl.BlockSpec((block_n, L), row),
            pl.BlockSpec((block_n, L), row),
            pl.BlockSpec((G, C), cst),
        ],
        out_shape=[
            jax.ShapeDtypeStruct((N, L), jnp.float32),
            jax.ShapeDtypeStruct((N, L), jnp.float32),
            jax.ShapeDtypeStruct((N, L), jnp.float32),
            jax.ShapeDtypeStruct((G, C), jnp.float32),
        ],
        scratch_shapes=[
            pltpu.VMEM((G, L), jnp.float32),
            pltpu.VMEM((G, 128), jnp.float32),
        ],
    )(x, *S0s, *S1s, *degs, batch_r,
      c1nmW, c1nmb, c1e2W, c1e2b, c2nmW, c2nmb, c2e2W, c2e2b,
      muW, mub, lvW, lvb, clsW, clsb)
    return out


def kernel(x, edge_index, edge_attr, batch, eps,
           c1e1W, c1e1b, c1e2W, c1e2b, c1nmW, c1nmb,
           c2e1W, c2e1b, c2e2W, c2e2b, c2nmW, c2nmb,
           muW, mub, lvW, lvb, clsW, clsb):
    N, D = x.shape
    E, ED = edge_attr.shape
    H = c1e1W.shape[1]
    dst = edge_index[1]

    Wcat = jnp.concatenate([c1e1W, c2e1W], axis=1).astype(jnp.float32)
    bcat = jnp.concatenate([c1e1b, c2e1b]).reshape(1, 2 * H).astype(jnp.float32)

    nh = (N + 127) // 128
    npad = 16 * NS
    Np = (N + npad - 1) // npad * npad
    zeros_init = jnp.zeros((Np, WC), jnp.float32)

    Eh = E // NSPLIT
    block_e = 3200
    dst_r = dst.reshape(E // block_e, 1, block_e)
    ea_t = edge_attr.T

    S0s, S1s, degs = [], [], []
    for p in range(NSPLIT):
        h1p, deg80 = _edge_mlp(ea_t, Wcat, bcat, dst_r, nh, p, NSPLIT, block_e)
        Sa, Sb = _sc_segsum(h1p, dst, zeros_init, p * Eh)
        S0s.append(Sa)
        S1s.append(Sb)
        degs.append(deg80.reshape(-1)[:N].reshape(N, 1))

    z, mu, lv, logits = _node_stage(
        x, S0s, S1s, degs, batch,
        c1nmW, c1nmb.reshape(1, H), c1e2W, c1e2b.reshape(1, H),
        c2nmW, c2nmb.reshape(1, H), c2e2W, c2e2b.reshape(1, H),
        muW, mub.reshape(1, -1), lvW, lvb.reshape(1, -1),
        clsW, clsb.reshape(1, -1))
    return (z, mu, lv, logits)


# speedup vs baseline: 8.4800x; 1.0331x over previous
import functools

import jax
import jax.numpy as jnp
from jax import lax
from jax.experimental import pallas as pl
from jax.experimental.pallas import tpu as pltpu
from jax.experimental.pallas import tpu_sc as plsc

NC, NS = 2, 16
WC = 128
K = 80
NSPLIT = 2


def _leaky(v):
    return jnp.where(v >= 0, v, 0.15 * v)


def _edge_mlp_body(ea_ref, w_ref, b_ref, ei_ref, out_ref, deg_ref, dst_ref,
                   deg_acc):
    i = pl.program_id(0)
    nb = pl.num_programs(0)
    f32 = jnp.float32
    v = lax.dot_general(ea_ref[...], w_ref[...], (((0,), (0,)), ((), ())),
                        preferred_element_type=f32)
    v = _leaky(v + b_ref[...])
    out_ref[0, :, :] = v[:, :WC]
    out_ref[1, :, :] = v[:, WC:]

    d = ei_ref[1, :]
    dst_ref[0, 0, :] = d
    be = d.shape[0]
    nh = deg_acc.shape[0]
    hi = d >> 7
    lo = d & 127
    oh_hi = (hi[:, None] == lax.broadcasted_iota(jnp.int32, (be, nh), 1)).astype(f32)
    oh_lo = (lo[:, None] == lax.broadcasted_iota(jnp.int32, (be, 128), 1)).astype(f32)

    @pl.when(i == 0)
    def _():
        deg_acc[...] = jnp.zeros_like(deg_acc)

    deg_acc[...] += lax.dot_general(
        oh_hi, oh_lo, (((0,), (0,)), ((), ())), preferred_element_type=f32)

    @pl.when(i == nb - 1)
    def _():
        deg_ref[...] = deg_acc[...]


def _edge_mlp(ea_t, Wcat, bcat, edge_index, nh, p, nsplit, block_e):
    ED, E = ea_t.shape
    Eh = E // nsplit
    W = Wcat.shape[1]
    nb = Eh // block_e
    off = p * nb
    return pl.pallas_call(
        _edge_mlp_body,
        grid=(nb,),
        in_specs=[
            pl.BlockSpec((ED, block_e), lambda i: (0, i + off)),
            pl.BlockSpec((ED, W), lambda i: (0, 0)),
            pl.BlockSpec((1, W), lambda i: (0, 0)),
            pl.BlockSpec((2, block_e), lambda i: (0, i + off)),
        ],
        out_specs=[
            pl.BlockSpec((NC, block_e, WC), lambda i: (0, i, 0)),
            pl.BlockSpec((nh, 128), lambda i: (0, 0)),
            pl.BlockSpec((1, 1, block_e), lambda i: (i, 0, 0)),
        ],
        out_shape=[
            jax.ShapeDtypeStruct((NC, Eh, WC), jnp.float32),
            jax.ShapeDtypeStruct((nh, 128), jnp.float32),
            jax.ShapeDtypeStruct((nb, 1, block_e), jnp.int32),
        ],
        scratch_shapes=[pltpu.VMEM((nh, 128), jnp.float32)],
    )(ea_t, Wcat, bcat, edge_index)


def _sc_segsum(h1p, dst, zeros_init, eoff):
    Eh = h1p.shape[1]
    Np = zeros_init.shape[0]
    ept = Eh // NS
    npt = Np // NS
    nchunks = ept // K
    nbuf = 2
    mesh = plsc.VectorSubcoreMesh(
        core_axis_name="c", subcore_axis_name="s",
        num_cores=NC, num_subcores=NS)

    @functools.partial(
        pl.kernel,
        out_type=[jax.ShapeDtypeStruct((Np, WC), jnp.float32),
                  jax.ShapeDtypeStruct((Np, WC), jnp.float32)],
        mesh=mesh,
        scratch_types=[
            pltpu.VMEM((nbuf, K), jnp.int32),
            pltpu.VMEM((nbuf, K, WC), jnp.float32),
            pltpu.VMEM_SHARED((Np, WC), jnp.float32),
            pltpu.SemaphoreType.DMA((nbuf,)),
            pltpu.SemaphoreType.DMA((nbuf,)),
        ],
    )
    def body(h1_hbm, dst_hbm, zero_hbm, out0_hbm, out1_hbm, idx_v, rows_v,
             acc, isem, rsem):
        c = lax.axis_index("c")
        s = lax.axis_index("s")
        r0 = s * npt
        pltpu.sync_copy(zero_hbm.at[pl.ds(r0, npt), :], acc.at[pl.ds(r0, npt), :])
        plsc.subcore_barrier()
        t0 = s * ept

        def load(e0, b):
            pltpu.async_copy(dst_hbm.at[pl.ds(eoff + e0, K)], idx_v.at[b],
                             isem.at[b])
            pltpu.async_copy(h1_hbm.at[c, pl.ds(e0, K), :], rows_v.at[b],
                             rsem.at[b])

        def consume(i, e0, b):
            pltpu.make_async_copy(dst_hbm.at[pl.ds(0, K)], idx_v.at[b],
                                  isem.at[b]).wait()
            pltpu.make_async_copy(h1_hbm.at[0, pl.ds(0, K), :],
                                  rows_v.at[b], rsem.at[b]).wait()
            pltpu.sync_copy(rows_v.at[b], acc.at[idx_v.at[b]], add=True)

            @pl.when(i + nbuf < nchunks)
            def _():
                load(e0 + nbuf * K, b)

        for b in range(min(nbuf, nchunks)):
            load(t0 + b * K, b)

        def outer(g, carry):
            for b in range(nbuf):
                i = g * nbuf + b
                consume(i, t0 + i * K, b)
            return carry

        lax.fori_loop(0, nchunks // nbuf, outer, 0)
        for b in range(nchunks % nbuf):
            i = (nchunks // nbuf) * nbuf + b
            consume(i, t0 + i * K, b)

        plsc.subcore_barrier()

        @pl.when(c == 0)
        def _():
            pltpu.sync_copy(acc.at[pl.ds(r0, npt), :],
                            out0_hbm.at[pl.ds(r0, npt), :])

        @pl.when(c == 1)
        def _():
            pltpu.sync_copy(acc.at[pl.ds(r0, npt), :],
                            out1_hbm.at[pl.ds(r0, npt), :])

    return body(h1p, dst, zeros_init)


def _node_body(x_ref, s0_refs, s1_refs, deg_refs, batch_ref,
               c1nmW_ref, c1nmb_ref, c1e2W_ref, c1e2b_ref,
               c2nmW_ref, c2nmb_ref, c2e2W_ref, c2e2b_ref,
               muW_ref, mub_ref, lvW_ref, lvb_ref, clsW_ref, clsb_ref,
               z_ref, mu_ref, lv_ref, logit_ref, pooled_acc, cnt_acc):
    i = pl.program_id(0)
    nb = pl.num_programs(0)
    x = x_ref[...]
    H = c1nmb_ref.shape[1]
    D = x.shape[1]
    S1 = s0_refs[0][...]
    for r in s0_refs[1:]:
        S1 = S1 + r[...]
    S2 = s1_refs[0][...]
    for r in s1_refs[1:]:
        S2 = S2 + r[...]
    deg = deg_refs[0][...]
    for r in deg_refs[1:]:
        deg = deg + r[...]

    f32 = jnp.float32
    w1 = c1nmW_ref[...]
    A1, Be1 = w1[:D], w1[D:]
    W2_1 = jnp.dot(c1e2W_ref[...], Be1, preferred_element_type=f32)
    u1 = jnp.dot(c1e2b_ref[...], Be1, preferred_element_type=f32) + c1nmb_ref[...]
    h = _leaky(deg * (jnp.dot(x, A1, preferred_element_type=f32) + u1)
               + jnp.dot(S1, W2_1, preferred_element_type=f32))

    w2 = c2nmW_ref[...]
    A2, Be2 = w2[:H], w2[H:]
    W2_2 = jnp.dot(c2e2W_ref[...], Be2, preferred_element_type=f32)
    u2 = jnp.dot(c2e2b_ref[...], Be2, preferred_element_type=f32) + c2nmb_ref[...]
    h2 = _leaky(deg * (jnp.dot(h, A2, preferred_element_type=f32) + u2)
                + jnp.dot(S2, W2_2, preferred_element_type=f32))

    mu = jnp.dot(h2, muW_ref[...], preferred_element_type=f32) + mub_ref[...]
    lv = jnp.dot(h2, lvW_ref[...], preferred_element_type=f32) + lvb_ref[...]
    z_ref[...] = mu
    mu_ref[...] = mu
    lv_ref[...] = lv

    G = pooled_acc.shape[0]
    rb = x.shape[0]
    b = batch_ref[0, 0, :]
    gids = lax.broadcasted_iota(jnp.int32, (rb, G), 1)
    oh = (b[:, None] == gids).astype(f32)

    @pl.when(i == 0)
    def _():
        pooled_acc[...] = jnp.zeros_like(pooled_acc)
        cnt_acc[...] = jnp.zeros_like(cnt_acc)

    pooled_acc[...] += lax.dot_general(
        oh, mu, (((0,), (0,)), ((), ())), preferred_element_type=f32)
    cnt_acc[...] += lax.dot_general(
        oh, jnp.ones((rb, 128), f32), (((0,), (0,)), ((), ())),
        preferred_element_type=f32)

    @pl.when(i == nb - 1)
    def _():
        cnt = jnp.maximum(cnt_acc[:, 0:1], 1.0)
        pooled = pooled_acc[...] / cnt
        logit_ref[...] = (jnp.dot(pooled, clsW_ref[...],
                                  preferred_element_type=f32)
                          + clsb_ref[...])


def _node_stage(x, S0s, S1s, degs, batch,
                c1nmW, c1nmb, c1e2W, c1e2b, c2nmW, c2nmb, c2e2W, c2e2b,
                muW, mub, lvW, lvb, clsW, clsb, block_n=2000):
    N, D = x.shape
    L = muW.shape[1]
    G, C = clsW.shape[0], clsW.shape[1]
    nb = N // block_n
    batch_r = batch.reshape(nb, 1, block_n)
    row = lambda i: (i, 0)
    cst = lambda i: (0, 0)
    full = lambda a: pl.BlockSpec(a.shape, cst)
    srow = pl.BlockSpec((block_n, WC), row)

    def wrapped(x_ref, *rest):
        np_ = len(S0s)
        s0r = rest[:np_]
        s1r = rest[np_:2 * np_]
        degr = rest[2 * np_:3 * np_]
        _node_body(x_ref, s0r, s1r, degr, *rest[3 * np_:])

    out = pl.pallas_call(
        wrapped,
        grid=(nb,),
        in_specs=[pl.BlockSpec((block_n, D), row)]
        + [srow] * len(S0s) + [srow] * len(S1s)
        + [pl.BlockSpec((block_n, 1), row)] * len(degs)
        + [pl.BlockSpec((1, 1, block_n), lambda i: (i, 0, 0)),
           full(c1nmW), full(c1nmb), full(c1e2W), full(c1e2b),
           full(c2nmW), full(c2nmb), full(c2e2W), full(c2e2b),
           full(muW), full(mub), full(lvW), full(lvb),
           full(clsW), full(clsb)],
        out_specs=[
            pl.BlockSpec((block_n, L), row),
            pl.BlockSpec((block_n, L), row),
            pl.BlockSpec((block_n, L), row),
            pl.BlockSpec((G, C), cst),
        ],
        out_shape=[
            jax.ShapeDtypeStruct((N, L), jnp.float32),
            jax.ShapeDtypeStruct((N, L), jnp.float32),
            jax.ShapeDtypeStruct((N, L), jnp.float32),
            jax.ShapeDtypeStruct((G, C), jnp.float32),
        ],
        scratch_shapes=[
            pltpu.VMEM((G, L), jnp.float32),
            pltpu.VMEM((G, 128), jnp.float32),
        ],
    )(x, *S0s, *S1s, *degs, batch_r,
      c1nmW, c1nmb, c1e2W, c1e2b, c2nmW, c2nmb, c2e2W, c2e2b,
      muW, mub, lvW, lvb, clsW, clsb)
    return out


def kernel(x, edge_index, edge_attr, batch, eps,
           c1e1W, c1e1b, c1e2W, c1e2b, c1nmW, c1nmb,
           c2e1W, c2e1b, c2e2W, c2e2b, c2nmW, c2nmb,
           muW, mub, lvW, lvb, clsW, clsb):
    N, D = x.shape
    E, ED = edge_attr.shape
    H = c1e1W.shape[1]

    Wcat = jnp.concatenate([c1e1W, c2e1W], axis=1).astype(jnp.float32)
    bcat = jnp.concatenate([c1e1b, c2e1b]).reshape(1, 2 * H).astype(jnp.float32)

    nh = (N + 127) // 128
    npad = 16 * NS
    Np = (N + npad - 1) // npad * npad
    zeros_init = jnp.zeros((Np, WC), jnp.float32)

    Eh = E // NSPLIT
    block_e = 3200
    ea_t = edge_attr.T

    S0s, S1s, degs = [], [], []
    for p in range(NSPLIT):
        h1p, deg80, dst_p = _edge_mlp(ea_t, Wcat, bcat, edge_index, nh, p,
                                      NSPLIT, block_e)
        Sa, Sb = _sc_segsum(h1p, dst_p.reshape(Eh), zeros_init, 0)
        S0s.append(Sa)
        S1s.append(Sb)
        degs.append(deg80.reshape(-1)[:N].reshape(N, 1))

    z, mu, lv, logits = _node_stage(
        x, S0s, S1s, degs, batch,
        c1nmW, c1nmb.reshape(1, H), c1e2W, c1e2b.reshape(1, H),
        c2nmW, c2nmb.reshape(1, H), c2e2W, c2e2b.reshape(1, H),
        muW, mub.reshape(1, -1), lvW, lvb.reshape(1, -1),
        clsW, clsb.reshape(1, -1))
    return (z, mu, lv, logits)


# speedup vs baseline: 8.5571x; 1.0091x over previous
import functools

import jax
import jax.numpy as jnp
from jax import lax
from jax.experimental import pallas as pl
from jax.experimental.pallas import tpu as pltpu
from jax.experimental.pallas import tpu_sc as plsc

NC, NS = 2, 16
WC = 128
K = 80
NSPLIT = 2


def _leaky(v):
    return jnp.where(v >= 0, v, 0.15 * v)


def _edge_mlp_body(ea_ref, w_ref, b_ref, ei_ref, out_ref, deg_ref, dst_ref,
                   deg_acc):
    i = pl.program_id(0)
    nb = pl.num_programs(0)
    f32 = jnp.float32
    v = lax.dot_general(ea_ref[...], w_ref[...], (((0,), (0,)), ((), ())),
                        preferred_element_type=f32)
    v = _leaky(v + b_ref[...])
    out_ref[0, :, :] = v[:, :WC]
    out_ref[1, :, :] = v[:, WC:]

    d = ei_ref[1, :]
    dst_ref[0, 0, :] = d
    be = d.shape[0]
    nh = deg_acc.shape[0]
    hi = d >> 7
    lo = d & 127
    bf = jnp.bfloat16
    oh_hi = (hi[:, None] == lax.broadcasted_iota(jnp.int32, (be, nh), 1)).astype(bf)
    oh_lo = (lo[:, None] == lax.broadcasted_iota(jnp.int32, (be, 128), 1)).astype(bf)

    @pl.when(i == 0)
    def _():
        deg_acc[...] = jnp.zeros_like(deg_acc)

    deg_acc[...] += lax.dot_general(
        oh_hi, oh_lo, (((0,), (0,)), ((), ())), preferred_element_type=f32)

    @pl.when(i == nb - 1)
    def _():
        deg_ref[...] = deg_acc[...]


def _edge_mlp(ea_t, Wcat, bcat, edge_index, nh, p, nsplit, block_e):
    ED, E = ea_t.shape
    Eh = E // nsplit
    W = Wcat.shape[1]
    nb = Eh // block_e
    off = p * nb
    return pl.pallas_call(
        _edge_mlp_body,
        grid=(nb,),
        in_specs=[
            pl.BlockSpec((ED, block_e), lambda i: (0, i + off)),
            pl.BlockSpec((ED, W), lambda i: (0, 0)),
            pl.BlockSpec((1, W), lambda i: (0, 0)),
            pl.BlockSpec((2, block_e), lambda i: (0, i + off)),
        ],
        out_specs=[
            pl.BlockSpec((NC, block_e, WC), lambda i: (0, i, 0)),
            pl.BlockSpec((nh, 128), lambda i: (0, 0)),
            pl.BlockSpec((1, 1, block_e), lambda i: (i, 0, 0)),
        ],
        out_shape=[
            jax.ShapeDtypeStruct((NC, Eh, WC), jnp.float32),
            jax.ShapeDtypeStruct((nh, 128), jnp.float32),
            jax.ShapeDtypeStruct((nb, 1, block_e), jnp.int32),
        ],
        scratch_shapes=[pltpu.VMEM((nh, 128), jnp.float32)],
    )(ea_t, Wcat, bcat, edge_index)


def _sc_segsum(h1p, dst, zeros_init, eoff):
    Eh = h1p.shape[1]
    Np = zeros_init.shape[0]
    ept = Eh // NS
    npt = Np // NS
    nchunks = ept // K
    nbuf = 2
    mesh = plsc.VectorSubcoreMesh(
        core_axis_name="c", subcore_axis_name="s",
        num_cores=NC, num_subcores=NS)

    @functools.partial(
        pl.kernel,
        out_type=[jax.ShapeDtypeStruct((Np, WC), jnp.float32),
                  jax.ShapeDtypeStruct((Np, WC), jnp.float32)],
        mesh=mesh,
        scratch_types=[
            pltpu.VMEM((nbuf, K), jnp.int32),
            pltpu.VMEM((nbuf, K, WC), jnp.float32),
            pltpu.VMEM_SHARED((Np, WC), jnp.float32),
            pltpu.SemaphoreType.DMA((nbuf,)),
            pltpu.SemaphoreType.DMA((nbuf,)),
        ],
    )
    def body(h1_hbm, dst_hbm, zero_hbm, out0_hbm, out1_hbm, idx_v, rows_v,
             acc, isem, rsem):
        c = lax.axis_index("c")
        s = lax.axis_index("s")
        r0 = s * npt
        pltpu.sync_copy(zero_hbm.at[pl.ds(r0, npt), :], acc.at[pl.ds(r0, npt), :])
        plsc.subcore_barrier()
        t0 = s * ept

        def load(e0, b):
            pltpu.async_copy(dst_hbm.at[pl.ds(eoff + e0, K)], idx_v.at[b],
                             isem.at[b])
            pltpu.async_copy(h1_hbm.at[c, pl.ds(e0, K), :], rows_v.at[b],
                             rsem.at[b])

        def consume(i, e0, b):
            pltpu.make_async_copy(dst_hbm.at[pl.ds(0, K)], idx_v.at[b],
                                  isem.at[b]).wait()
            pltpu.make_async_copy(h1_hbm.at[0, pl.ds(0, K), :],
                                  rows_v.at[b], rsem.at[b]).wait()
            pltpu.sync_copy(rows_v.at[b], acc.at[idx_v.at[b]], add=True)

            @pl.when(i + nbuf < nchunks)
            def _():
                load(e0 + nbuf * K, b)

        for b in range(min(nbuf, nchunks)):
            load(t0 + b * K, b)

        def outer(g, carry):
            for b in range(nbuf):
                i = g * nbuf + b
                consume(i, t0 + i * K, b)
            return carry

        lax.fori_loop(0, nchunks // nbuf, outer, 0)
        for b in range(nchunks % nbuf):
            i = (nchunks // nbuf) * nbuf + b
            consume(i, t0 + i * K, b)

        plsc.subcore_barrier()

        @pl.when(c == 0)
        def _():
            pltpu.sync_copy(acc.at[pl.ds(r0, npt), :],
                            out0_hbm.at[pl.ds(r0, npt), :])

        @pl.when(c == 1)
        def _():
            pltpu.sync_copy(acc.at[pl.ds(r0, npt), :],
                            out1_hbm.at[pl.ds(r0, npt), :])

    return body(h1p, dst, zeros_init)


def _node_body(x_ref, s0_refs, s1_refs, deg_refs, batch_ref,
               c1nmW_ref, c1nmb_ref, c1e2W_ref, c1e2b_ref,
               c2nmW_ref, c2nmb_ref, c2e2W_ref, c2e2b_ref,
               muW_ref, mub_ref, lvW_ref, lvb_ref, clsW_ref, clsb_ref,
               z_ref, mu_ref, lv_ref, logit_ref, pooled_acc, cnt_acc):
    i = pl.program_id(0)
    nb = pl.num_programs(0)
    x = x_ref[...]
    H = c1nmb_ref.shape[1]
    D = x.shape[1]
    S1 = s0_refs[0][...]
    for r in s0_refs[1:]:
        S1 = S1 + r[...]
    S2 = s1_refs[0][...]
    for r in s1_refs[1:]:
        S2 = S2 + r[...]
    deg = deg_refs[0][...]
    for r in deg_refs[1:]:
        deg = deg + r[...]

    f32 = jnp.float32
    w1 = c1nmW_ref[...]
    A1, Be1 = w1[:D], w1[D:]
    W2_1 = jnp.dot(c1e2W_ref[...], Be1, preferred_element_type=f32)
    u1 = jnp.dot(c1e2b_ref[...], Be1, preferred_element_type=f32) + c1nmb_ref[...]
    h = _leaky(deg * (jnp.dot(x, A1, preferred_element_type=f32) + u1)
               + jnp.dot(S1, W2_1, preferred_element_type=f32))

    w2 = c2nmW_ref[...]
    A2, Be2 = w2[:H], w2[H:]
    W2_2 = jnp.dot(c2e2W_ref[...], Be2, preferred_element_type=f32)
    u2 = jnp.dot(c2e2b_ref[...], Be2, preferred_element_type=f32) + c2nmb_ref[...]
    h2 = _leaky(deg * (jnp.dot(h, A2, preferred_element_type=f32) + u2)
                + jnp.dot(S2, W2_2, preferred_element_type=f32))

    mu = jnp.dot(h2, muW_ref[...], preferred_element_type=f32) + mub_ref[...]
    lv = jnp.dot(h2, lvW_ref[...], preferred_element_type=f32) + lvb_ref[...]
    z_ref[...] = mu
    mu_ref[...] = mu
    lv_ref[...] = lv

    G = pooled_acc.shape[0]
    rb = x.shape[0]
    b = batch_ref[0, 0, :]
    gids = lax.broadcasted_iota(jnp.int32, (rb, G), 1)
    oh = (b[:, None] == gids).astype(f32)

    @pl.when(i == 0)
    def _():
        pooled_acc[...] = jnp.zeros_like(pooled_acc)
        cnt_acc[...] = jnp.zeros_like(cnt_acc)

    pooled_acc[...] += lax.dot_general(
        oh, mu, (((0,), (0,)), ((), ())), preferred_element_type=f32)
    cnt_acc[...] += lax.dot_general(
        oh, jnp.ones((rb, 128), f32), (((0,), (0,)), ((), ())),
        preferred_element_type=f32)

    @pl.when(i == nb - 1)
    def _():
        cnt = jnp.maximum(cnt_acc[:, 0:1], 1.0)
        pooled = pooled_acc[...] / cnt
        logit_ref[...] = (jnp.dot(pooled, clsW_ref[...],
                                  preferred_element_type=f32)
                          + clsb_ref[...])


def _node_stage(x, S0s, S1s, degs, batch,
                c1nmW, c1nmb, c1e2W, c1e2b, c2nmW, c2nmb, c2e2W, c2e2b,
                muW, mub, lvW, lvb, clsW, clsb, block_n=2000):
    N, D = x.shape
    L = muW.shape[1]
    G, C = clsW.shape[0], clsW.shape[1]
    nb = N // block_n
    batch_r = batch.reshape(nb, 1, block_n)
    row = lambda i: (i, 0)
    cst = lambda i: (0, 0)
    full = lambda a: pl.BlockSpec(a.shape, cst)
    srow = pl.BlockSpec((block_n, WC), row)

    def wrapped(x_ref, *rest):
        np_ = len(S0s)
        s0r = rest[:np_]
        s1r = rest[np_:2 * np_]
        degr = rest[2 * np_:3 * np_]
        _node_body(x_ref, s0r, s1r, degr, *rest[3 * np_:])

    out = pl.pallas_call(
        wrapped,
        grid=(nb,),
        in_specs=[pl.BlockSpec((block_n, D), row)]
        + [srow] * len(S0s) + [srow] * len(S1s)
        + [pl.BlockSpec((block_n, 1), row)] * len(degs)
        + [pl.BlockSpec((1, 1, block_n), lambda i: (i, 0, 0)),
           full(c1nmW), full(c1nmb), full(c1e2W), full(c1e2b),
           full(c2nmW), full(c2nmb), full(c2e2W), full(c2e2b),
           full(muW), full(mub), full(lvW), full(lvb),
           full(clsW), full(clsb)],
        out_specs=[
            pl.BlockSpec((block_n, L), row),
            pl.BlockSpec((block_n, L), row),
            pl.BlockSpec((block_n, L), row),
            pl.BlockSpec((G, C), cst),
        ],
        out_shape=[
            jax.ShapeDtypeStruct((N, L), jnp.float32),
            jax.ShapeDtypeStruct((N, L), jnp.float32),
            jax.ShapeDtypeStruct((N, L), jnp.float32),
            jax.ShapeDtypeStruct((G, C), jnp.float32),
        ],
        scratch_shapes=[
            pltpu.VMEM((G, L), jnp.float32),
            pltpu.VMEM((G, 128), jnp.float32),
        ],
    )(x, *S0s, *S1s, *degs, batch_r,
      c1nmW, c1nmb, c1e2W, c1e2b, c2nmW, c2nmb, c2e2W, c2e2b,
      muW, mub, lvW, lvb, clsW, clsb)
    return out


def kernel(x, edge_index, edge_attr, batch, eps,
           c1e1W, c1e1b, c1e2W, c1e2b, c1nmW, c1nmb,
           c2e1W, c2e1b, c2e2W, c2e2b, c2nmW, c2nmb,
           muW, mub, lvW, lvb, clsW, clsb):
    N, D = x.shape
    E, ED = edge_attr.shape
    H = c1e1W.shape[1]

    Wcat = jnp.concatenate([c1e1W, c2e1W], axis=1).astype(jnp.float32)
    bcat = jnp.concatenate([c1e1b, c2e1b]).reshape(1, 2 * H).astype(jnp.float32)

    nh = (N + 127) // 128
    npad = 16 * NS
    Np = (N + npad - 1) // npad * npad
    zeros_init = jnp.zeros((Np, WC), jnp.float32)

    Eh = E // NSPLIT
    block_e = 6400
    ea_t = edge_attr.T

    S0s, S1s, degs = [], [], []
    for p in range(NSPLIT):
        h1p, deg80, dst_p = _edge_mlp(ea_t, Wcat, bcat, edge_index, nh, p,
                                      NSPLIT, block_e)
        Sa, Sb = _sc_segsum(h1p, dst_p.reshape(Eh), zeros_init, 0)
        S0s.append(Sa)
        S1s.append(Sb)
        degs.append(deg80.reshape(-1)[:N].reshape(N, 1))

    z, mu, lv, logits = _node_stage(
        x, S0s, S1s, degs, batch,
        c1nmW, c1nmb.reshape(1, H), c1e2W, c1e2b.reshape(1, H),
        c2nmW, c2nmb.reshape(1, H), c2e2W, c2e2b.reshape(1, H),
        muW, mub.reshape(1, -1), lvW, lvb.reshape(1, -1),
        clsW, clsb.reshape(1, -1))
    return (z, mu, lv, logits)


# speedup vs baseline: 8.5676x; 1.0012x over previous
import functools

import jax
import jax.numpy as jnp
from jax import lax
from jax.experimental import pallas as pl
from jax.experimental.pallas import tpu as pltpu
from jax.experimental.pallas import tpu_sc as plsc

NC, NS = 2, 16
WC = 128
K = 80
NSPLIT = 2
NG = 64


def _leaky(v):
    return jnp.where(v >= 0, v, 0.15 * v)


def _edge_mlp_body(ea_ref, w_ref, b_ref, ei_ref, out_ref, deg_ref, dst_ref,
                   deg_acc):
    i = pl.program_id(0)
    nb = pl.num_programs(0)
    f32 = jnp.float32
    v = lax.dot_general(ea_ref[...], w_ref[...], (((0,), (0,)), ((), ())),
                        preferred_element_type=f32)
    v = _leaky(v + b_ref[...])
    out_ref[0, :, :] = v[:, :WC]
    out_ref[1, :, :] = v[:, WC:]

    d = ei_ref[1, :]
    dst_ref[0, 0, :] = d
    be = d.shape[0]
    nh = deg_acc.shape[0]
    hi = d >> 7
    lo = d & 127
    bf = jnp.bfloat16
    oh_hi = (hi[:, None] == lax.broadcasted_iota(jnp.int32, (be, nh), 1)).astype(bf)
    oh_lo = (lo[:, None] == lax.broadcasted_iota(jnp.int32, (be, 128), 1)).astype(bf)

    @pl.when(i == 0)
    def _():
        deg_acc[...] = jnp.zeros_like(deg_acc)

    deg_acc[...] += lax.dot_general(
        oh_hi, oh_lo, (((0,), (0,)), ((), ())), preferred_element_type=f32)

    @pl.when(i == nb - 1)
    def _():
        deg_ref[...] = deg_acc[...]


def _edge_mlp(ea_t, Wcat, bcat, edge_index, nh, p, nsplit, block_e):
    ED, E = ea_t.shape
    Eh = E // nsplit
    W = Wcat.shape[1]
    nb = Eh // block_e
    off = p * nb
    return pl.pallas_call(
        _edge_mlp_body,
        grid=(nb,),
        in_specs=[
            pl.BlockSpec((ED, block_e), lambda i: (0, i + off)),
            pl.BlockSpec((ED, W), lambda i: (0, 0)),
            pl.BlockSpec((1, W), lambda i: (0, 0)),
            pl.BlockSpec((2, block_e), lambda i: (0, i + off)),
        ],
        out_specs=[
            pl.BlockSpec((NC, block_e, WC), lambda i: (0, i, 0)),
            pl.BlockSpec((nh, 128), lambda i: (0, 0)),
            pl.BlockSpec((1, 1, block_e), lambda i: (i, 0, 0)),
        ],
        out_shape=[
            jax.ShapeDtypeStruct((NC, Eh, WC), jnp.float32),
            jax.ShapeDtypeStruct((nh, 128), jnp.float32),
            jax.ShapeDtypeStruct((nb, 1, block_e), jnp.int32),
        ],
        scratch_shapes=[pltpu.VMEM((nh, 128), jnp.float32)],
    )(ea_t, Wcat, bcat, edge_index)


def _sc_segsum(h1p, dst, zeros_init, eoff):
    Eh = h1p.shape[1]
    Np = zeros_init.shape[0]
    ept = Eh // NS
    npt = Np // NS
    nchunks = ept // K
    nbuf = 2
    mesh = plsc.VectorSubcoreMesh(
        core_axis_name="c", subcore_axis_name="s",
        num_cores=NC, num_subcores=NS)

    @functools.partial(
        pl.kernel,
        out_type=[jax.ShapeDtypeStruct((Np, WC), jnp.float32),
                  jax.ShapeDtypeStruct((Np, WC), jnp.float32)],
        mesh=mesh,
        scratch_types=[
            pltpu.VMEM((nbuf, K), jnp.int32),
            pltpu.VMEM((nbuf, K, WC), jnp.float32),
            pltpu.VMEM_SHARED((Np, WC), jnp.float32),
            pltpu.SemaphoreType.DMA((nbuf,)),
            pltpu.SemaphoreType.DMA((nbuf,)),
        ],
    )
    def body(h1_hbm, dst_hbm, zero_hbm, out0_hbm, out1_hbm, idx_v, rows_v,
             acc, isem, rsem):
        c = lax.axis_index("c")
        s = lax.axis_index("s")
        r0 = s * npt
        pltpu.sync_copy(zero_hbm.at[pl.ds(r0, npt), :], acc.at[pl.ds(r0, npt), :])
        plsc.subcore_barrier()
        t0 = s * ept

        def load(e0, b):
            pltpu.async_copy(dst_hbm.at[pl.ds(eoff + e0, K)], idx_v.at[b],
                             isem.at[b])
            pltpu.async_copy(h1_hbm.at[c, pl.ds(e0, K), :], rows_v.at[b],
                             rsem.at[b])

        def consume(i, e0, b):
            pltpu.make_async_copy(dst_hbm.at[pl.ds(0, K)], idx_v.at[b],
                                  isem.at[b]).wait()
            pltpu.make_async_copy(h1_hbm.at[0, pl.ds(0, K), :],
                                  rows_v.at[b], rsem.at[b]).wait()
            pltpu.sync_copy(rows_v.at[b], acc.at[idx_v.at[b]], add=True)

            @pl.when(i + nbuf < nchunks)
            def _():
                load(e0 + nbuf * K, b)

        for b in range(min(nbuf, nchunks)):
            load(t0 + b * K, b)

        def outer(g, carry):
            for b in range(nbuf):
                i = g * nbuf + b
                consume(i, t0 + i * K, b)
            return carry

        lax.fori_loop(0, nchunks // nbuf, outer, 0)
        for b in range(nchunks % nbuf):
            i = (nchunks // nbuf) * nbuf + b
            consume(i, t0 + i * K, b)

        plsc.subcore_barrier()

        @pl.when(c == 0)
        def _():
            pltpu.sync_copy(acc.at[pl.ds(r0, npt), :],
                            out0_hbm.at[pl.ds(r0, npt), :])

        @pl.when(c == 1)
        def _():
            pltpu.sync_copy(acc.at[pl.ds(r0, npt), :],
                            out1_hbm.at[pl.ds(r0, npt), :])

    return body(h1p, dst, zeros_init)


def _node_body(x_ref, s0_refs, s1_refs, deg_refs, batch_ref,
               c1nmW_ref, c1nmb_ref, c1e2W_ref, c1e2b_ref,
               c2nmW_ref, c2nmb_ref, c2e2W_ref, c2e2b_ref,
               muW_ref, mub_ref, lvW_ref, lvb_ref, clsW_ref, clsb_ref,
               z_ref, mu_ref, lv_ref, logit_ref, pooled_acc, cnt_acc):
    i = pl.program_id(0)
    nb = pl.num_programs(0)
    x = x_ref[...]
    H = c1nmb_ref.shape[1]
    D = x.shape[1]
    S1 = s0_refs[0][...]
    for r in s0_refs[1:]:
        S1 = S1 + r[...]
    S2 = s1_refs[0][...]
    for r in s1_refs[1:]:
        S2 = S2 + r[...]
    deg = deg_refs[0][...]
    for r in deg_refs[1:]:
        deg = deg + r[...]

    f32 = jnp.float32
    w1 = c1nmW_ref[...]
    A1, Be1 = w1[:D], w1[D:]
    W2_1 = jnp.dot(c1e2W_ref[...], Be1, preferred_element_type=f32)
    u1 = jnp.dot(c1e2b_ref[...], Be1, preferred_element_type=f32) + c1nmb_ref[...]
    h = _leaky(deg * (jnp.dot(x, A1, preferred_element_type=f32) + u1)
               + jnp.dot(S1, W2_1, preferred_element_type=f32))

    w2 = c2nmW_ref[...]
    A2, Be2 = w2[:H], w2[H:]
    W2_2 = jnp.dot(c2e2W_ref[...], Be2, preferred_element_type=f32)
    u2 = jnp.dot(c2e2b_ref[...], Be2, preferred_element_type=f32) + c2nmb_ref[...]
    h2 = _leaky(deg * (jnp.dot(h, A2, preferred_element_type=f32) + u2)
                + jnp.dot(S2, W2_2, preferred_element_type=f32))

    mu = jnp.dot(h2, muW_ref[...], preferred_element_type=f32) + mub_ref[...]
    lv = jnp.dot(h2, lvW_ref[...], preferred_element_type=f32) + lvb_ref[...]
    z_ref[...] = mu
    mu_ref[...] = mu
    lv_ref[...] = lv

    G = pooled_acc.shape[1]
    rb = x.shape[0]
    b = batch_ref[0, 0, :]
    gids = lax.broadcasted_iota(jnp.int32, (rb, G), 1)
    oh = (b[:, None] == gids).astype(f32)

    @pl.when(i == 0)
    def _():
        pooled_acc[...] = jnp.zeros_like(pooled_acc)
        cnt_acc[...] = jnp.zeros_like(cnt_acc)

    pooled_acc[...] += lax.dot_general(
        mu, oh, (((0,), (0,)), ((), ())), preferred_element_type=f32)
    cnt_acc[...] += lax.dot_general(
        jnp.ones((rb, 8), f32), oh, (((0,), (0,)), ((), ())),
        preferred_element_type=f32)

    @pl.when(i == nb - 1)
    def _():
        cnt = jnp.maximum(cnt_acc[0:1, :], 1.0)
        pooledT = pooled_acc[...] / cnt
        logit_ref[...] = (lax.dot_general(
            clsW_ref[...], pooledT, (((0,), (0,)), ((), ())),
            preferred_element_type=f32) + clsb_ref[...])


def _node_stage(x, S0s, S1s, degs, batch,
                c1nmW, c1nmb, c1e2W, c1e2b, c2nmW, c2nmb, c2e2W, c2e2b,
                muW, mub, lvW, lvb, clsW, clsb, block_n=2000):
    N, D = x.shape
    L = muW.shape[1]
    C = clsW.shape[1]
    G = NG
    nb = N // block_n
    batch_r = batch.reshape(nb, 1, block_n)
    row = lambda i: (i, 0)
    cst = lambda i: (0, 0)
    full = lambda a: pl.BlockSpec(a.shape, cst)
    srow = pl.BlockSpec((block_n, WC), row)

    def wrapped(x_ref, *rest):
        np_ = len(S0s)
        s0r = rest[:np_]
        s1r = rest[np_:2 * np_]
        degr = rest[2 * np_:3 * np_]
        _node_body(x_ref, s0r, s1r, degr, *rest[3 * np_:])

    out = pl.pallas_call(
        wrapped,
        grid=(nb,),
        in_specs=[pl.BlockSpec((block_n, D), row)]
        + [srow] * len(S0s) + [srow] * len(S1s)
        + [pl.BlockSpec((block_n, 1), row)] * len(degs)
        + [pl.BlockSpec((1, 1, block_n), lambda i: (i, 0, 0)),
           full(c1nmW), full(c1nmb), full(c1e2W), full(c1e2b),
           full(c2nmW), full(c2nmb), full(c2e2W), full(c2e2b),
           full(muW), full(mub), full(lvW), full(lvb),
           full(clsW), full(clsb)],
        out_specs=[
            pl.BlockSpec((block_n, L), row),
            pl.BlockSpec((block_n, L), row),
            pl.BlockSpec((block_n, L), row),
            pl.BlockSpec((C, G), cst),
        ],
        out_shape=[
            jax.ShapeDtypeStruct((N, L), jnp.float32),
            jax.ShapeDtypeStruct((N, L), jnp.float32),
            jax.ShapeDtypeStruct((N, L), jnp.float32),
            jax.ShapeDtypeStruct((C, G), jnp.float32),
        ],
        scratch_shapes=[
            pltpu.VMEM((L, G), jnp.float32),
            pltpu.VMEM((8, G), jnp.float32),
        ],
    )(x, *S0s, *S1s, *degs, batch_r,
      c1nmW, c1nmb, c1e2W, c1e2b, c2nmW, c2nmb, c2e2W, c2e2b,
      muW, mub, lvW, lvb, clsW, clsb)
    return out


def kernel(x, edge_index, edge_attr, batch, eps,
           c1e1W, c1e1b, c1e2W, c1e2b, c1nmW, c1nmb,
           c2e1W, c2e1b, c2e2W, c2e2b, c2nmW, c2nmb,
           muW, mub, lvW, lvb, clsW, clsb):
    N, D = x.shape
    E, ED = edge_attr.shape
    H = c1e1W.shape[1]

    Wcat = jnp.concatenate([c1e1W, c2e1W], axis=1).astype(jnp.float32)
    bcat = jnp.concatenate([c1e1b, c2e1b]).reshape(1, 2 * H).astype(jnp.float32)

    nh = (N + 127) // 128
    npad = 16 * NS
    Np = (N + npad - 1) // npad * npad
    zeros_init = jnp.zeros((Np, WC), jnp.float32)

    Eh = E // NSPLIT
    block_e = 6400
    ea_t = edge_attr.T

    S0s, S1s, degs = [], [], []
    for p in range(NSPLIT):
        h1p, deg80, dst_p = _edge_mlp(ea_t, Wcat, bcat, edge_index, nh, p,
                                      NSPLIT, block_e)
        Sa, Sb = _sc_segsum(h1p, dst_p.reshape(Eh), zeros_init, 0)
        S0s.append(Sa)
        S1s.append(Sb)
        degs.append(deg80.reshape(-1)[:N].reshape(N, 1))

    z, mu, lv, logitsT = _node_stage(
        x, S0s, S1s, degs, batch,
        c1nmW, c1nmb.reshape(1, H), c1e2W, c1e2b.reshape(1, H),
        c2nmW, c2nmb.reshape(1, H), c2e2W, c2e2b.reshape(1, H),
        muW, mub.reshape(1, -1), lvW, lvb.reshape(1, -1),
        clsW, clsb.reshape(-1, 1))
    return (z, mu, lv, logitsT.T)


# speedup vs baseline: 9.3452x; 1.0908x over previous
import functools

import jax
import jax.numpy as jnp
from jax import lax
from jax.experimental import pallas as pl
from jax.experimental.pallas import tpu as pltpu
from jax.experimental.pallas import tpu_sc as plsc

NC, NS = 2, 16
WC = 128
K = 80
NSPLIT = 2
NG = 64


def _leaky(v):
    return jnp.where(v >= 0, v, 0.15 * v)


def _edge_mlp_body(ea_ref, w_ref, b_ref, ei_ref, out_ref, deg_ref, dst_ref,
                   deg_acc):
    i = pl.program_id(0)
    nb = pl.num_programs(0)
    f32 = jnp.float32
    v = lax.dot_general(ea_ref[...], w_ref[...], (((0,), (0,)), ((), ())),
                        preferred_element_type=f32)
    v = _leaky(v + b_ref[...])
    out_ref[0, :, :] = v[:, :WC]
    out_ref[1, :, :] = v[:, WC:]

    d = ei_ref[1, :]
    dst_ref[0, 0, :] = d
    be = d.shape[0]
    nh = deg_acc.shape[0]
    hi = d >> 7
    lo = d & 127
    bf = jnp.bfloat16
    oh_hi = (hi[:, None] == lax.broadcasted_iota(jnp.int32, (be, nh), 1)).astype(bf)
    oh_lo = (lo[:, None] == lax.broadcasted_iota(jnp.int32, (be, 128), 1)).astype(bf)

    @pl.when(i == 0)
    def _():
        deg_acc[...] = jnp.zeros_like(deg_acc)

    deg_acc[...] += lax.dot_general(
        oh_hi, oh_lo, (((0,), (0,)), ((), ())), preferred_element_type=f32)

    @pl.when(i == nb - 1)
    def _():
        deg_ref[...] = deg_acc[...]


def _edge_mlp(ea_t, Wcat, bcat, edge_index, nh, p, nsplit, block_e):
    ED, E = ea_t.shape
    Eh = E // nsplit
    W = Wcat.shape[1]
    nb = Eh // block_e
    off = p * nb
    return pl.pallas_call(
        _edge_mlp_body,
        grid=(nb,),
        in_specs=[
            pl.BlockSpec((ED, block_e), lambda i: (0, i + off)),
            pl.BlockSpec((ED, W), lambda i: (0, 0)),
            pl.BlockSpec((1, W), lambda i: (0, 0)),
            pl.BlockSpec((2, block_e), lambda i: (0, i + off)),
        ],
        out_specs=[
            pl.BlockSpec((NC, block_e, WC), lambda i: (0, i, 0)),
            pl.BlockSpec((nh, 128), lambda i: (0, 0)),
            pl.BlockSpec((1, 1, block_e), lambda i: (i, 0, 0)),
        ],
        out_shape=[
            jax.ShapeDtypeStruct((NC, Eh, WC), jnp.float32),
            jax.ShapeDtypeStruct((nh, 128), jnp.float32),
            jax.ShapeDtypeStruct((nb, 1, block_e), jnp.int32),
        ],
        scratch_shapes=[pltpu.VMEM((nh, 128), jnp.float32)],
    )(ea_t, Wcat, bcat, edge_index)


def _sc_segsum(h1p, dst, zeros_init, eoff):
    Eh = h1p.shape[1]
    Np = zeros_init.shape[0]
    ept = Eh // NS
    npt = Np // NS
    nchunks = ept // K
    nbuf = 3
    mesh = plsc.VectorSubcoreMesh(
        core_axis_name="c", subcore_axis_name="s",
        num_cores=NC, num_subcores=NS)

    @functools.partial(
        pl.kernel,
        out_type=[jax.ShapeDtypeStruct((Np, WC), jnp.float32),
                  jax.ShapeDtypeStruct((Np, WC), jnp.float32)],
        mesh=mesh,
        scratch_types=[
            pltpu.VMEM((nbuf, K), jnp.int32),
            pltpu.VMEM((nbuf, K, WC), jnp.float32),
            pltpu.VMEM_SHARED((Np, WC), jnp.float32),
            pltpu.SemaphoreType.DMA((nbuf,)),
            pltpu.SemaphoreType.DMA((nbuf,)),
        ],
    )
    def body(h1_hbm, dst_hbm, zero_hbm, out0_hbm, out1_hbm, idx_v, rows_v,
             acc, isem, rsem):
        c = lax.axis_index("c")
        s = lax.axis_index("s")
        r0 = s * npt
        pltpu.sync_copy(zero_hbm.at[pl.ds(r0, npt), :], acc.at[pl.ds(r0, npt), :])
        plsc.subcore_barrier()
        t0 = s * ept

        def load(e0, b):
            pltpu.async_copy(dst_hbm.at[pl.ds(eoff + e0, K)], idx_v.at[b],
                             isem.at[b])
            pltpu.async_copy(h1_hbm.at[c, pl.ds(e0, K), :], rows_v.at[b],
                             rsem.at[b])

        def consume(i, e0, b):
            pltpu.make_async_copy(dst_hbm.at[pl.ds(0, K)], idx_v.at[b],
                                  isem.at[b]).wait()
            pltpu.make_async_copy(h1_hbm.at[0, pl.ds(0, K), :],
                                  rows_v.at[b], rsem.at[b]).wait()
            pltpu.sync_copy(rows_v.at[b], acc.at[idx_v.at[b]], add=True)

            @pl.when(i + nbuf < nchunks)
            def _():
                load(e0 + nbuf * K, b)

        for b in range(min(nbuf, nchunks)):
            load(t0 + b * K, b)

        def outer(g, carry):
            for b in range(nbuf):
                i = g * nbuf + b
                consume(i, t0 + i * K, b)
            return carry

        lax.fori_loop(0, nchunks // nbuf, outer, 0)
        for b in range(nchunks % nbuf):
            i = (nchunks // nbuf) * nbuf + b
            consume(i, t0 + i * K, b)

        plsc.subcore_barrier()

        @pl.when(c == 0)
        def _():
            pltpu.sync_copy(acc.at[pl.ds(r0, npt), :],
                            out0_hbm.at[pl.ds(r0, npt), :])

        @pl.when(c == 1)
        def _():
            pltpu.sync_copy(acc.at[pl.ds(r0, npt), :],
                            out1_hbm.at[pl.ds(r0, npt), :])

    return body(h1p, dst, zeros_init)


def _node_body(x_ref, s0_refs, s1_refs, deg_refs, batch_ref,
               c1nmW_ref, c1nmb_ref, c1e2W_ref, c1e2b_ref,
               c2nmW_ref, c2nmb_ref, c2e2W_ref, c2e2b_ref,
               muW_ref, mub_ref, lvW_ref, lvb_ref, clsW_ref, clsb_ref,
               z_ref, mu_ref, lv_ref, logit_ref, pooled_acc, cnt_acc):
    i = pl.program_id(0)
    nb = pl.num_programs(0)
    x = x_ref[...]
    H = c1nmb_ref.shape[1]
    D = x.shape[1]
    S1 = s0_refs[0][...]
    for r in s0_refs[1:]:
        S1 = S1 + r[...]
    S2 = s1_refs[0][...]
    for r in s1_refs[1:]:
        S2 = S2 + r[...]
    deg = deg_refs[0][...]
    for r in deg_refs[1:]:
        deg = deg + r[...]

    f32 = jnp.float32
    w1 = c1nmW_ref[...]
    A1, Be1 = w1[:D], w1[D:]
    W2_1 = jnp.dot(c1e2W_ref[...], Be1, preferred_element_type=f32)
    u1 = jnp.dot(c1e2b_ref[...], Be1, preferred_element_type=f32) + c1nmb_ref[...]
    h = _leaky(deg * (jnp.dot(x, A1, preferred_element_type=f32) + u1)
               + jnp.dot(S1, W2_1, preferred_element_type=f32))

    w2 = c2nmW_ref[...]
    A2, Be2 = w2[:H], w2[H:]
    W2_2 = jnp.dot(c2e2W_ref[...], Be2, preferred_element_type=f32)
    u2 = jnp.dot(c2e2b_ref[...], Be2, preferred_element_type=f32) + c2nmb_ref[...]
    h2 = _leaky(deg * (jnp.dot(h, A2, preferred_element_type=f32) + u2)
                + jnp.dot(S2, W2_2, preferred_element_type=f32))

    mu = jnp.dot(h2, muW_ref[...], preferred_element_type=f32) + mub_ref[...]
    lv = jnp.dot(h2, lvW_ref[...], preferred_element_type=f32) + lvb_ref[...]
    z_ref[...] = mu
    mu_ref[...] = mu
    lv_ref[...] = lv

    G = pooled_acc.shape[1]
    rb = x.shape[0]
    b = batch_ref[0, 0, :]
    gids = lax.broadcasted_iota(jnp.int32, (rb, G), 1)
    oh = (b[:, None] == gids).astype(f32)

    @pl.when(i == 0)
    def _():
        pooled_acc[...] = jnp.zeros_like(pooled_acc)
        cnt_acc[...] = jnp.zeros_like(cnt_acc)

    pooled_acc[...] += lax.dot_general(
        mu, oh, (((0,), (0,)), ((), ())), preferred_element_type=f32)
    cnt_acc[...] += lax.dot_general(
        jnp.ones((rb, 8), f32), oh, (((0,), (0,)), ((), ())),
        preferred_element_type=f32)

    @pl.when(i == nb - 1)
    def _():
        cnt = jnp.maximum(cnt_acc[0:1, :], 1.0)
        pooledT = pooled_acc[...] / cnt
        logit_ref[...] = (lax.dot_general(
            clsW_ref[...], pooledT, (((0,), (0,)), ((), ())),
            preferred_element_type=f32) + clsb_ref[...])


def _node_stage(x, S0s, S1s, degs, batch,
                c1nmW, c1nmb, c1e2W, c1e2b, c2nmW, c2nmb, c2e2W, c2e2b,
                muW, mub, lvW, lvb, clsW, clsb, block_n=2000):
    N, D = x.shape
    L = muW.shape[1]
    C = clsW.shape[1]
    G = NG
    nb = N // block_n
    batch_r = batch.reshape(nb, 1, block_n)
    row = lambda i: (i, 0)
    cst = lambda i: (0, 0)
    full = lambda a: pl.BlockSpec(a.shape, cst)
    srow = pl.BlockSpec((block_n, WC), row)

    def wrapped(x_ref, *rest):
        np_ = len(S0s)
        s0r = rest[:np_]
        s1r = rest[np_:2 * np_]
        degr = rest[2 * np_:3 * np_]
        _node_body(x_ref, s0r, s1r, degr, *rest[3 * np_:])

    out = pl.pallas_call(
        wrapped,
        grid=(nb,),
        in_specs=[pl.BlockSpec((block_n, D), row)]
        + [srow] * len(S0s) + [srow] * len(S1s)
        + [pl.BlockSpec((block_n, 1), row)] * len(degs)
        + [pl.BlockSpec((1, 1, block_n), lambda i: (i, 0, 0)),
           full(c1nmW), full(c1nmb), full(c1e2W), full(c1e2b),
           full(c2nmW), full(c2nmb), full(c2e2W), full(c2e2b),
           full(muW), full(mub), full(lvW), full(lvb),
           full(clsW), full(clsb)],
        out_specs=[
            pl.BlockSpec((block_n, L), row),
            pl.BlockSpec((block_n, L), row),
            pl.BlockSpec((block_n, L), row),
            pl.BlockSpec((C, G), cst),
        ],
        out_shape=[
            jax.ShapeDtypeStruct((N, L), jnp.float32),
            jax.ShapeDtypeStruct((N, L), jnp.float32),
            jax.ShapeDtypeStruct((N, L), jnp.float32),
            jax.ShapeDtypeStruct((C, G), jnp.float32),
        ],
        scratch_shapes=[
            pltpu.VMEM((L, G), jnp.float32),
            pltpu.VMEM((8, G), jnp.float32),
        ],
    )(x, *S0s, *S1s, *degs, batch_r,
      c1nmW, c1nmb, c1e2W, c1e2b, c2nmW, c2nmb, c2e2W, c2e2b,
      muW, mub, lvW, lvb, clsW, clsb)
    return out


def kernel(x, edge_index, edge_attr, batch, eps,
           c1e1W, c1e1b, c1e2W, c1e2b, c1nmW, c1nmb,
           c2e1W, c2e1b, c2e2W, c2e2b, c2nmW, c2nmb,
           muW, mub, lvW, lvb, clsW, clsb):
    N, D = x.shape
    E, ED = edge_attr.shape
    H = c1e1W.shape[1]

    Wcat = jnp.concatenate([c1e1W, c2e1W], axis=1).astype(jnp.float32)
    bcat = jnp.concatenate([c1e1b, c2e1b]).reshape(1, 2 * H).astype(jnp.float32)

    nh = (N + 127) // 128
    npad = 16 * NS
    Np = (N + npad - 1) // npad * npad
    zeros_init = jnp.zeros((Np, WC), jnp.float32)

    Eh = E // NSPLIT
    block_e = 6400
    ea_t = edge_attr.T

    S0s, S1s, degs = [], [], []
    for p in range(NSPLIT):
        h1p, deg80, dst_p = _edge_mlp(ea_t, Wcat, bcat, edge_index, nh, p,
                                      NSPLIT, block_e)
        Sa, Sb = _sc_segsum(h1p, dst_p.reshape(Eh), zeros_init, 0)
        S0s.append(Sa)
        S1s.append(Sb)
        degs.append(deg80.reshape(-1)[:N].reshape(N, 1))

    z, mu, lv, logitsT = _node_stage(
        x, S0s, S1s, degs, batch,
        c1nmW, c1nmb.reshape(1, H), c1e2W, c1e2b.reshape(1, H),
        c2nmW, c2nmb.reshape(1, H), c2e2W, c2e2b.reshape(1, H),
        muW, mub.reshape(1, -1), lvW, lvb.reshape(1, -1),
        clsW, clsb.reshape(-1, 1))
    return (z, mu, lv, logitsT.T)


# speedup vs baseline: 9.3716x; 1.0028x over previous
import functools

import jax
import jax.numpy as jnp
from jax import lax
from jax.experimental import pallas as pl
from jax.experimental.pallas import tpu as pltpu
from jax.experimental.pallas import tpu_sc as plsc

NC, NS = 2, 16
WC = 128
K = 80
NSPLIT = 2
NG = 64


def _leaky(v):
    return jnp.where(v >= 0, v, 0.15 * v)


def _edge_mlp_body(ea_ref, w_ref, b_ref, ei_ref, out_ref, deg_ref, dst_ref,
                   deg_acc):
    i = pl.program_id(0)
    nb = pl.num_programs(0)
    f32 = jnp.float32
    v = lax.dot_general(ea_ref[...], w_ref[...], (((0,), (0,)), ((), ())),
                        preferred_element_type=f32)
    v = _leaky(v + b_ref[...])
    out_ref[0, :, :] = v[:, :WC]
    out_ref[1, :, :] = v[:, WC:]

    d = ei_ref[1, :]
    dst_ref[0, 0, :] = d
    be = d.shape[0]
    nh = deg_acc.shape[0]
    hi = d >> 7
    lo = d & 127
    bf = jnp.bfloat16
    oh_hi = (hi[:, None] == lax.broadcasted_iota(jnp.int32, (be, nh), 1)).astype(bf)
    oh_lo = (lo[:, None] == lax.broadcasted_iota(jnp.int32, (be, 128), 1)).astype(bf)

    @pl.when(i == 0)
    def _():
        deg_acc[...] = jnp.zeros_like(deg_acc)

    deg_acc[...] += lax.dot_general(
        oh_hi, oh_lo, (((0,), (0,)), ((), ())), preferred_element_type=f32)

    @pl.when(i == nb - 1)
    def _():
        deg_ref[...] = deg_acc[...]


def _edge_mlp(ea_t, Wcat, bcat, edge_index, nh, p, nsplit, block_e):
    ED, E = ea_t.shape
    Eh = E // nsplit
    W = Wcat.shape[1]
    nb = Eh // block_e
    off = p * nb
    return pl.pallas_call(
        _edge_mlp_body,
        grid=(nb,),
        in_specs=[
            pl.BlockSpec((ED, block_e), lambda i: (0, i + off)),
            pl.BlockSpec((ED, W), lambda i: (0, 0)),
            pl.BlockSpec((1, W), lambda i: (0, 0)),
            pl.BlockSpec((2, block_e), lambda i: (0, i + off)),
        ],
        out_specs=[
            pl.BlockSpec((NC, block_e, WC), lambda i: (0, i, 0)),
            pl.BlockSpec((nh, 128), lambda i: (0, 0)),
            pl.BlockSpec((1, 1, block_e), lambda i: (i, 0, 0)),
        ],
        out_shape=[
            jax.ShapeDtypeStruct((NC, Eh, WC), jnp.float32),
            jax.ShapeDtypeStruct((nh, 128), jnp.float32),
            jax.ShapeDtypeStruct((nb, 1, block_e), jnp.int32),
        ],
        scratch_shapes=[pltpu.VMEM((nh, 128), jnp.float32)],
    )(ea_t, Wcat, bcat, edge_index)


def _sc_segsum(h1p, dst, zeros_init, eoff):
    Eh = h1p.shape[1]
    Np = zeros_init.shape[0]
    ept = Eh // NS
    npt = Np // NS
    nchunks = ept // K
    nbuf = 4
    mesh = plsc.VectorSubcoreMesh(
        core_axis_name="c", subcore_axis_name="s",
        num_cores=NC, num_subcores=NS)

    @functools.partial(
        pl.kernel,
        out_type=[jax.ShapeDtypeStruct((Np, WC), jnp.float32),
                  jax.ShapeDtypeStruct((Np, WC), jnp.float32)],
        mesh=mesh,
        scratch_types=[
            pltpu.VMEM((nbuf, K), jnp.int32),
            pltpu.VMEM((nbuf, K, WC), jnp.float32),
            pltpu.VMEM_SHARED((Np, WC), jnp.float32),
            pltpu.SemaphoreType.DMA((nbuf,)),
            pltpu.SemaphoreType.DMA((nbuf,)),
        ],
    )
    def body(h1_hbm, dst_hbm, zero_hbm, out0_hbm, out1_hbm, idx_v, rows_v,
             acc, isem, rsem):
        c = lax.axis_index("c")
        s = lax.axis_index("s")
        r0 = s * npt
        pltpu.sync_copy(zero_hbm.at[pl.ds(r0, npt), :], acc.at[pl.ds(r0, npt), :])
        plsc.subcore_barrier()
        t0 = s * ept

        def load(e0, b):
            pltpu.async_copy(dst_hbm.at[pl.ds(eoff + e0, K)], idx_v.at[b],
                             isem.at[b])
            pltpu.async_copy(h1_hbm.at[c, pl.ds(e0, K), :], rows_v.at[b],
                             rsem.at[b])

        def consume(i, e0, b):
            pltpu.make_async_copy(dst_hbm.at[pl.ds(0, K)], idx_v.at[b],
                                  isem.at[b]).wait()
            pltpu.make_async_copy(h1_hbm.at[0, pl.ds(0, K), :],
                                  rows_v.at[b], rsem.at[b]).wait()
            pltpu.sync_copy(rows_v.at[b], acc.at[idx_v.at[b]], add=True)

            @pl.when(i + nbuf < nchunks)
            def _():
                load(e0 + nbuf * K, b)

        for b in range(min(nbuf, nchunks)):
            load(t0 + b * K, b)

        def outer(g, carry):
            for b in range(nbuf):
                i = g * nbuf + b
                consume(i, t0 + i * K, b)
            return carry

        lax.fori_loop(0, nchunks // nbuf, outer, 0)
        for b in range(nchunks % nbuf):
            i = (nchunks // nbuf) * nbuf + b
            consume(i, t0 + i * K, b)

        plsc.subcore_barrier()

        @pl.when(c == 0)
        def _():
            pltpu.sync_copy(acc.at[pl.ds(r0, npt), :],
                            out0_hbm.at[pl.ds(r0, npt), :])

        @pl.when(c == 1)
        def _():
            pltpu.sync_copy(acc.at[pl.ds(r0, npt), :],
                            out1_hbm.at[pl.ds(r0, npt), :])

    return body(h1p, dst, zeros_init)


def _node_body(x_ref, s0_refs, s1_refs, deg_refs, batch_ref,
               c1nmW_ref, c1nmb_ref, c1e2W_ref, c1e2b_ref,
               c2nmW_ref, c2nmb_ref, c2e2W_ref, c2e2b_ref,
               muW_ref, mub_ref, lvW_ref, lvb_ref, clsW_ref, clsb_ref,
               z_ref, mu_ref, lv_ref, logit_ref, pooled_acc, cnt_acc):
    i = pl.program_id(0)
    nb = pl.num_programs(0)
    x = x_ref[...]
    H = c1nmb_ref.shape[1]
    D = x.shape[1]
    S1 = s0_refs[0][...]
    for r in s0_refs[1:]:
        S1 = S1 + r[...]
    S2 = s1_refs[0][...]
    for r in s1_refs[1:]:
        S2 = S2 + r[...]
    deg = deg_refs[0][...]
    for r in deg_refs[1:]:
        deg = deg + r[...]

    f32 = jnp.float32
    w1 = c1nmW_ref[...]
    A1, Be1 = w1[:D], w1[D:]
    W2_1 = jnp.dot(c1e2W_ref[...], Be1, preferred_element_type=f32)
    u1 = jnp.dot(c1e2b_ref[...], Be1, preferred_element_type=f32) + c1nmb_ref[...]
    h = _leaky(deg * (jnp.dot(x, A1, preferred_element_type=f32) + u1)
               + jnp.dot(S1, W2_1, preferred_element_type=f32))

    w2 = c2nmW_ref[...]
    A2, Be2 = w2[:H], w2[H:]
    W2_2 = jnp.dot(c2e2W_ref[...], Be2, preferred_element_type=f32)
    u2 = jnp.dot(c2e2b_ref[...], Be2, preferred_element_type=f32) + c2nmb_ref[...]
    h2 = _leaky(deg * (jnp.dot(h, A2, preferred_element_type=f32) + u2)
                + jnp.dot(S2, W2_2, preferred_element_type=f32))

    mu = jnp.dot(h2, muW_ref[...], preferred_element_type=f32) + mub_ref[...]
    lv = jnp.dot(h2, lvW_ref[...], preferred_element_type=f32) + lvb_ref[...]
    z_ref[...] = mu
    mu_ref[...] = mu
    lv_ref[...] = lv

    G = pooled_acc.shape[1]
    rb = x.shape[0]
    b = batch_ref[0, 0, :]
    gids = lax.broadcasted_iota(jnp.int32, (rb, G), 1)
    oh = (b[:, None] == gids).astype(f32)

    @pl.when(i == 0)
    def _():
        pooled_acc[...] = jnp.zeros_like(pooled_acc)
        cnt_acc[...] = jnp.zeros_like(cnt_acc)

    pooled_acc[...] += lax.dot_general(
        mu, oh, (((0,), (0,)), ((), ())), preferred_element_type=f32)
    cnt_acc[...] += lax.dot_general(
        jnp.ones((rb, 8), f32), oh, (((0,), (0,)), ((), ())),
        preferred_element_type=f32)

    @pl.when(i == nb - 1)
    def _():
        cnt = jnp.maximum(cnt_acc[0:1, :], 1.0)
        pooledT = pooled_acc[...] / cnt
        logit_ref[...] = (lax.dot_general(
            clsW_ref[...], pooledT, (((0,), (0,)), ((), ())),
            preferred_element_type=f32) + clsb_ref[...])


def _node_stage(x, S0s, S1s, degs, batch,
                c1nmW, c1nmb, c1e2W, c1e2b, c2nmW, c2nmb, c2e2W, c2e2b,
                muW, mub, lvW, lvb, clsW, clsb, block_n=2000):
    N, D = x.shape
    L = muW.shape[1]
    C = clsW.shape[1]
    G = NG
    nb = N // block_n
    batch_r = batch.reshape(nb, 1, block_n)
    row = lambda i: (i, 0)
    cst = lambda i: (0, 0)
    full = lambda a: pl.BlockSpec(a.shape, cst)
    srow = pl.BlockSpec((block_n, WC), row)

    def wrapped(x_ref, *rest):
        np_ = len(S0s)
        s0r = rest[:np_]
        s1r = rest[np_:2 * np_]
        degr = rest[2 * np_:3 * np_]
        _node_body(x_ref, s0r, s1r, degr, *rest[3 * np_:])

    out = pl.pallas_call(
        wrapped,
        grid=(nb,),
        in_specs=[pl.BlockSpec((block_n, D), row)]
        + [srow] * len(S0s) + [srow] * len(S1s)
        + [pl.BlockSpec((block_n, 1), row)] * len(degs)
        + [pl.BlockSpec((1, 1, block_n), lambda i: (i, 0, 0)),
           full(c1nmW), full(c1nmb), full(c1e2W), full(c1e2b),
           full(c2nmW), full(c2nmb), full(c2e2W), full(c2e2b),
           full(muW), full(mub), full(lvW), full(lvb),
           full(clsW), full(clsb)],
        out_specs=[
            pl.BlockSpec((block_n, L), row),
            pl.BlockSpec((block_n, L), row),
            pl.BlockSpec((block_n, L), row),
            pl.BlockSpec((C, G), cst),
        ],
        out_shape=[
            jax.ShapeDtypeStruct((N, L), jnp.float32),
            jax.ShapeDtypeStruct((N, L), jnp.float32),
            jax.ShapeDtypeStruct((N, L), jnp.float32),
            jax.ShapeDtypeStruct((C, G), jnp.float32),
        ],
        scratch_shapes=[
            pltpu.VMEM((L, G), jnp.float32),
            pltpu.VMEM((8, G), jnp.float32),
        ],
    )(x, *S0s, *S1s, *degs, batch_r,
      c1nmW, c1nmb, c1e2W, c1e2b, c2nmW, c2nmb, c2e2W, c2e2b,
      muW, mub, lvW, lvb, clsW, clsb)
    return out


def kernel(x, edge_index, edge_attr, batch, eps,
           c1e1W, c1e1b, c1e2W, c1e2b, c1nmW, c1nmb,
           c2e1W, c2e1b, c2e2W, c2e2b, c2nmW, c2nmb,
           muW, mub, lvW, lvb, clsW, clsb):
    N, D = x.shape
    E, ED = edge_attr.shape
    H = c1e1W.shape[1]

    Wcat = jnp.concatenate([c1e1W, c2e1W], axis=1).astype(jnp.float32)
    bcat = jnp.concatenate([c1e1b, c2e1b]).reshape(1, 2 * H).astype(jnp.float32)

    nh = (N + 127) // 128
    npad = 16 * NS
    Np = (N + npad - 1) // npad * npad
    zeros_init = jnp.zeros((Np, WC), jnp.float32)

    Eh = E // NSPLIT
    block_e = 6400
    ea_t = edge_attr.T

    S0s, S1s, degs = [], [], []
    for p in range(NSPLIT):
        h1p, deg80, dst_p = _edge_mlp(ea_t, Wcat, bcat, edge_index, nh, p,
                                      NSPLIT, block_e)
        Sa, Sb = _sc_segsum(h1p, dst_p.reshape(Eh), zeros_init, 0)
        S0s.append(Sa)
        S1s.append(Sb)
        degs.append(deg80.reshape(-1)[:N].reshape(N, 1))

    z, mu, lv, logitsT = _node_stage(
        x, S0s, S1s, degs, batch,
        c1nmW, c1nmb.reshape(1, H), c1e2W, c1e2b.reshape(1, H),
        c2nmW, c2nmb.reshape(1, H), c2e2W, c2e2b.reshape(1, H),
        muW, mub.reshape(1, -1), lvW, lvb.reshape(1, -1),
        clsW, clsb.reshape(-1, 1))
    return (z, mu, lv, logitsT.T)
